# Initial kernel scaffold; baseline (speedup 1.0000x reference)
import jax
import jax.numpy as jnp
from jax.experimental import pallas as pl


def kernel(x, edge_index, batch, W1, a_src1, a_dst1, b1, g1, be1, W2, a_src2, a_dst2, b2, g2, be2, W3, a_src3, a_dst3, b3, g3, be3, Wout, bout):
    raise NotImplementedError("write your pallas kernel here")


# speedup vs baseline: 1.0553x; 1.0553x over previous
import jax
import jax.numpy as jnp
from jax.experimental import pallas as pl

N = 50000
E = 800000
H = 4
C = 32
HID = 128
G = 64
NEG_SLOPE = 0.2
EPS_BN = 1e-5


def _gat_layer(x, src, dst, W, a_src, a_dst, b):
    n = x.shape[0]
    h = (x @ W).reshape(n, H, C)
    alpha_s = (h * a_src[None, :, :]).sum(-1)
    alpha_d = (h * a_dst[None, :, :]).sum(-1)
    e = alpha_s[src] + alpha_d[dst]
    e = jnp.where(e > 0, e, NEG_SLOPE * e)
    ex = jnp.exp(e)
    denom = jax.ops.segment_sum(ex, dst, num_segments=n)
    msg = h[src] * ex[:, :, None]
    out = jax.ops.segment_sum(msg, dst, num_segments=n)
    out = out / (denom[:, :, None] + 1e-16)
    return out.reshape(n, H * C) + b[None, :]


def _bn_relu(x, g, b):
    mu = x.mean(axis=0)
    var = x.var(axis=0)
    return jax.nn.relu(g * (x - mu) / jnp.sqrt(var + EPS_BN) + b)


def _final_matmul_kernel(gr_ref, w_ref, b_ref, o_ref):
    o_ref[...] = gr_ref[...] @ w_ref[...] + b_ref[...]


def kernel(x, edge_index, batch, W1, a_src1, a_dst1, b1, g1, be1, W2, a_src2, a_dst2, b2, g2, be2, W3, a_src3, a_dst3, b3, g3, be3, Wout, bout):
    src = edge_index[0]
    dst = edge_index[1]
    h = x
    for (W, a_s, a_d, b, g, be) in [
        (W1, a_src1, a_dst1, b1, g1, be1),
        (W2, a_src2, a_dst2, b2, g2, be2),
        (W3, a_src3, a_dst3, b3, g3, be3),
    ]:
        h = _gat_layer(h, src, dst, W, a_s, a_d, b)
        h = _bn_relu(h, g, be)
    ones = jnp.ones((h.shape[0],), jnp.float32)
    counts = jax.ops.segment_sum(ones, batch, num_segments=G)
    mean_pool = jax.ops.segment_sum(h, batch, num_segments=G) / jnp.maximum(counts, 1.0)[:, None]
    max_pool = jax.ops.segment_max(h, batch, num_segments=G)
    max_pool = jnp.where(jnp.isfinite(max_pool), max_pool, 0.0)
    graph_repr = jnp.concatenate([mean_pool, max_pool], axis=1)
    out = pl.pallas_call(
        _final_matmul_kernel,
        out_shape=jax.ShapeDtypeStruct((G, HID), jnp.float32),
    )(graph_repr, Wout, bout[None, :])
    return out


# speedup vs baseline: 28.5066x; 27.0134x over previous
import dataclasses
import functools
import jax
import jax.numpy as jnp
from jax import lax
from jax.experimental import pallas as pl
from jax.experimental.pallas import tpu as pltpu
from jax.experimental.pallas import tpu_sc as plsc

N = 50000
E = 800000
H = 4
C = 32
HID = 128
G = 64
NEG_SLOPE = 0.2
EPS_BN = 1e-5

NSC = 2
NTI = 16
NW = NSC * NTI
EPT = 25600
EPAD = NW * EPT
ROWS = EPAD // 128
RPT = EPT // 128
CHUNK_ROWS = 8
NCHUNK = RPT // CHUNK_ROWS
SL = 3128
SL_LAST = N - 15 * SL

BN = 512
NBLK = (N + BN - 1) // BN
NPAD = NBLK * BN

_mesh = plsc.VectorSubcoreMesh(core_axis_name="c", subcore_axis_name="s")

_cp = pltpu.CompilerParams()
if "needs_layout_passes" in pltpu.CompilerParams.__dataclass_fields__:
    _cp = dataclasses.replace(_cp, needs_layout_passes=False)
if "use_tc_tiling_on_sc" in pltpu.CompilerParams.__dataclass_fields__:
    _cp = dataclasses.replace(_cp, use_tc_tiling_on_sc=False)


def _iota16():
    return lax.iota(jnp.int32, 16)


def _splat16(v):
    return lax.broadcast(v, (16,))


def _sc_pass1(src2d, dst2d, asv, adv, zeros1):
    def body(src_hbm, dst_hbm, as_hbm, ad_hbm, zeros_hbm, w_hbm, dpart_hbm,
             as_tab, ad_tab, src_buf, dst_buf, w_buf, dzero, dstage, dacc,
             sem):
        cid = lax.axis_index("c")
        sid = lax.axis_index("s")
        wid = cid * NTI + sid
        tr0 = wid * RPT

        pltpu.sync_copy(zeros_hbm.at[pl.ds(0, 3200)], dzero)

        for h in range(H):
            pltpu.sync_copy(as_hbm.at[pl.ds(h * N, N)], as_tab)
            pltpu.sync_copy(ad_hbm.at[pl.ds(h * N, N)], ad_tab)
            @pl.when(sid < NTI - 1)
            def _():
                pltpu.sync_copy(dzero.at[pl.ds(0, SL)],
                                dacc.at[pl.ds(sid * SL, SL)])
            @pl.when(sid == NTI - 1)
            def _():
                pltpu.sync_copy(dzero.at[pl.ds(0, SL_LAST)],
                                dacc.at[pl.ds(sid * SL, SL_LAST)])
            plsc.subcore_barrier()

            @pl.loop(0, NCHUNK)
            def _chunk(ch):
                r0 = tr0 + ch * CHUNK_ROWS
                pltpu.sync_copy(src_hbm.at[pl.ds(r0, CHUNK_ROWS)], src_buf)
                pltpu.sync_copy(dst_hbm.at[pl.ds(r0, CHUNK_ROWS)], dst_buf)

                @pl.loop(0, CHUNK_ROWS * 8)
                def _grp(g):
                    row = g >> 3
                    off = (g & 7) * 16
                    sv = src_buf[row, pl.ds(off, 16)]
                    dv = dst_buf[row, pl.ds(off, 16)]
                    es = plsc.load_gather(as_tab, [sv])
                    ed = plsc.load_gather(ad_tab, [dv])
                    e = es + ed
                    e = jnp.where(e > 0, e, NEG_SLOPE * e)
                    wv = jnp.exp(e)
                    eid = _splat16(r0 * 128 + g * 16) + _iota16()
                    wv = jnp.where(eid < E, wv, 0.0)
                    w_buf[row, pl.ds(off, 16)] = wv

                pltpu.sync_copy(w_buf, w_hbm.at[h, pl.ds(r0, CHUNK_ROWS)])
                for j in range(CHUNK_ROWS):
                    pltpu.sync_copy(w_buf.at[j], dacc.at[dst_buf.at[j]],
                                    add=True)

            plsc.subcore_barrier()
            dbase = cid * (H * N) + h * N + sid * SL
            @pl.when(sid < NTI - 1)
            def _():
                pltpu.sync_copy(dacc.at[pl.ds(sid * SL, SL)],
                                dstage.at[pl.ds(0, SL)])
                pltpu.sync_copy(dstage.at[pl.ds(0, SL)],
                                dpart_hbm.at[pl.ds(dbase, SL)])
            @pl.when(sid == NTI - 1)
            def _():
                pltpu.sync_copy(dacc.at[pl.ds(sid * SL, SL_LAST)],
                                dstage.at[pl.ds(0, SL_LAST)])
                pltpu.sync_copy(dstage.at[pl.ds(0, SL_LAST)],
                                dpart_hbm.at[pl.ds(dbase, SL_LAST)])
            plsc.subcore_barrier()

    k = pl.kernel(
        body,
        out_type=[
            jax.ShapeDtypeStruct((H, ROWS, 128), jnp.float32),
            jax.ShapeDtypeStruct((NSC * H * N,), jnp.float32),
        ],
        mesh=_mesh,
        scratch_types=[
            pltpu.VMEM((N,), jnp.float32),
            pltpu.VMEM((N,), jnp.float32),
            pltpu.VMEM((CHUNK_ROWS, 128), jnp.int32),
            pltpu.VMEM((CHUNK_ROWS, 128), jnp.int32),
            pltpu.VMEM((CHUNK_ROWS, 128), jnp.float32),
            pltpu.VMEM((3200,), jnp.float32),
            pltpu.VMEM((SL,), jnp.float32),
            pltpu.VMEM_SHARED((N,), jnp.float32),
            pltpu.SemaphoreType.DMA,
        ],
        compiler_params=_cp,
    )
    return k(src2d, dst2d, asv, adv, zeros1)


def _sc_alpha(dst2d, w, denom):
    def body(dst_hbm, w_hbm, den_hbm, a_hbm,
             den_tab, dst_buf, w_buf, a_buf, sem):
        cid = lax.axis_index("c")
        sid = lax.axis_index("s")
        wid = cid * NTI + sid
        tr0 = wid * RPT

        for h in range(H):
            pltpu.sync_copy(den_hbm.at[pl.ds(h * N, N)], den_tab)

            @pl.loop(0, RPT // 8)
            def _chunk(ch):
                r0 = tr0 + ch * 8
                pltpu.sync_copy(dst_hbm.at[pl.ds(r0, 8)], dst_buf)
                pltpu.sync_copy(w_hbm.at[h, pl.ds(r0, 8)], w_buf)

                @pl.loop(0, 64)
                def _grp(g):
                    row = g >> 3
                    off = (g & 7) * 16
                    dv_idx = dst_buf[row, pl.ds(off, 16)]
                    dvals = plsc.load_gather(den_tab, [dv_idx])
                    wv = w_buf[row, pl.ds(off, 16)]
                    a_buf[row, pl.ds(off, 16)] = wv / (dvals + 1e-16)

                pltpu.sync_copy(a_buf, a_hbm.at[h, pl.ds(r0, 8)])

    k = pl.kernel(
        body,
        out_type=jax.ShapeDtypeStruct((H, ROWS, 128), jnp.float32),
        mesh=_mesh,
        scratch_types=[
            pltpu.VMEM((N,), jnp.float32),
            pltpu.VMEM((8, 128), jnp.int32),
            pltpu.VMEM((8, 128), jnp.float32),
            pltpu.VMEM((8, 128), jnp.float32),
            pltpu.SemaphoreType.DMA,
        ],
        compiler_params=_cp,
    )
    return k(dst2d, w, denom)


P2R = 4
P2CHUNKS = RPT // P2R


def _sc_pass2(src2d, dst2d, alpha, htabs, zeros_o):
    def body(src_hbm, dst_hbm, a_hbm, h0, h1, h2, h3, zeros_hbm,
             op0, op1, op2, op3,
             src_buf, dst_buf, a_buf, rows3d, sbuf, oacc, sem):
        opart_list = [op0, op1, op2, op3]
        cid = lax.axis_index("c")
        sid = lax.axis_index("s")
        wid = cid * NTI + sid
        tr0 = wid * RPT
        htab_list = [h0, h1, h2, h3]

        for h in range(H):
            htab = htab_list[h]
            base = sid * SL
            @pl.when(sid < NTI - 1)
            def _():
                pltpu.sync_copy(zeros_hbm.at[pl.ds(0, SL)],
                                oacc.at[pl.ds(base, SL)])
            @pl.when(sid == NTI - 1)
            def _():
                pltpu.sync_copy(zeros_hbm.at[pl.ds(0, SL_LAST)],
                                oacc.at[pl.ds(base, SL_LAST)])
            plsc.subcore_barrier()

            @pl.loop(0, P2CHUNKS)
            def _chunk(ch):
                r0 = tr0 + ch * P2R
                pltpu.sync_copy(src_hbm.at[pl.ds(r0, P2R)], src_buf)
                pltpu.sync_copy(dst_hbm.at[pl.ds(r0, P2R)], dst_buf)
                pltpu.sync_copy(a_hbm.at[h, pl.ds(r0, P2R)], a_buf)

                cps = []
                for j in range(P2R):
                    cps.append(pltpu.async_copy(
                        htab.at[src_buf.at[j]], rows3d.at[j], sem))
                for cp in cps:
                    cp.wait()

                for j in range(P2R):
                    @pl.loop(0, 128)
                    def _r(r):
                        sv = plsc.load_gather(a_buf,
                                              [_splat16(j), _splat16(r)])
                        rows3d[j, r, pl.ds(0, 16)] = (
                            rows3d[j, r, pl.ds(0, 16)] * sv)
                        rows3d[j, r, pl.ds(16, 16)] = (
                            rows3d[j, r, pl.ds(16, 16)] * sv)

                for j in range(P2R):
                    pltpu.sync_copy(rows3d.at[j],
                                    oacc.at[dst_buf.at[j]], add=True)

            plsc.subcore_barrier()
            opart_hbm = opart_list[h]
            for kk in range(35):
                pltpu.sync_copy(oacc.at[pl.ds(base + kk * 88, 88)], sbuf)
                pltpu.sync_copy(sbuf,
                                opart_hbm.at[cid, pl.ds(base + kk * 88, 88)])
            @pl.when(sid < NTI - 1)
            def _():
                t = SL - 35 * 88
                pltpu.sync_copy(oacc.at[pl.ds(base + 35 * 88, t)],
                                sbuf.at[pl.ds(0, t)])
                pltpu.sync_copy(sbuf.at[pl.ds(0, t)],
                                opart_hbm.at[cid, pl.ds(base + 35 * 88, t)])
            plsc.subcore_barrier()

    k = pl.kernel(
        body,
        out_type=[jax.ShapeDtypeStruct((NSC, N, C), jnp.float32)
                  for _ in range(H)],
        mesh=_mesh,
        scratch_types=[
            pltpu.VMEM((P2R, 128), jnp.int32),
            pltpu.VMEM((P2R, 128), jnp.int32),
            pltpu.VMEM((P2R, 128), jnp.float32),
            pltpu.VMEM((P2R, 128, C), jnp.float32),
            pltpu.VMEM((88, C), jnp.float32),
            pltpu.VMEM_SHARED((N, C), jnp.float32),
            pltpu.SemaphoreType.DMA,
        ],
        compiler_params=_cp,
    )
    return k(src2d, dst2d, alpha, *htabs, zeros_o)


def _embed_body(x_ref, w_ref, asf_ref, adf_ref,
                h0_ref, h1_ref, h2_ref, h3_ref, as_ref, ad_ref):
    hblk = jax.lax.dot_general(
        x_ref[...], w_ref[...], (((1,), (0,)), ((), ())),
        preferred_element_type=jnp.float32)
    for h, o in enumerate((h0_ref, h1_ref, h2_ref, h3_ref)):
        o[...] = hblk[:, h * C:(h + 1) * C]
    as_ref[...] = jax.lax.dot_general(
        asf_ref[...], hblk, (((1,), (1,)), ((), ())),
        preferred_element_type=jnp.float32)
    ad_ref[...] = jax.lax.dot_general(
        adf_ref[...], hblk, (((1,), (1,)), ((), ())),
        preferred_element_type=jnp.float32)


def _k_embed(hin, W, a_src, a_dst):
    din = hin.shape[1]
    eye4 = jnp.eye(H, dtype=jnp.float32)
    asf = (a_src[:, None, :] * eye4[:, :, None]).reshape(H, H * C)
    adf = (a_dst[:, None, :] * eye4[:, :, None]).reshape(H, H * C)
    outs = pl.pallas_call(
        _embed_body,
        grid=(NBLK,),
        in_specs=[
            pl.BlockSpec((BN, din), lambda i: (i, 0)),
            pl.BlockSpec((din, H * C), lambda i: (0, 0)),
            pl.BlockSpec((H, H * C), lambda i: (0, 0)),
            pl.BlockSpec((H, H * C), lambda i: (0, 0)),
        ],
        out_specs=[
            pl.BlockSpec((BN, C), lambda i: (i, 0)),
            pl.BlockSpec((BN, C), lambda i: (i, 0)),
            pl.BlockSpec((BN, C), lambda i: (i, 0)),
            pl.BlockSpec((BN, C), lambda i: (i, 0)),
            pl.BlockSpec((H, BN), lambda i: (0, i)),
            pl.BlockSpec((H, BN), lambda i: (0, i)),
        ],
        out_shape=[
            jax.ShapeDtypeStruct((N, C), jnp.float32),
            jax.ShapeDtypeStruct((N, C), jnp.float32),
            jax.ShapeDtypeStruct((N, C), jnp.float32),
            jax.ShapeDtypeStruct((N, C), jnp.float32),
            jax.ShapeDtypeStruct((H, N), jnp.float32),
            jax.ShapeDtypeStruct((H, N), jnp.float32),
        ],
    )(hin, W, asf, adf)
    return outs[:4], outs[4], outs[5]


def _dsum_body(d_ref, o_ref):
    o_ref[...] = d_ref[0] + d_ref[1]


def _k_dsum(dpart):
    return pl.pallas_call(
        _dsum_body,
        grid=(NBLK,),
        in_specs=[pl.BlockSpec((NSC, H, BN), lambda i: (0, 0, i))],
        out_specs=pl.BlockSpec((H, BN), lambda i: (0, i)),
        out_shape=jax.ShapeDtypeStruct((H, N), jnp.float32),
    )(dpart)


def _comb_body(p0_ref, p1_ref, p2_ref, p3_ref, b_ref, h_ref, st_ref,
               ssum, ssq):
    i = pl.program_id(0)
    hpre = jnp.concatenate(
        [p[0] + p[1] for p in (p0_ref, p1_ref, p2_ref, p3_ref)],
        axis=1) + b_ref[...]
    h_ref[...] = hpre
    valid = (i * BN + lax.broadcasted_iota(jnp.int32, (BN, 1), 0)) < N
    hm = jnp.where(valid, hpre, 0.0)

    @pl.when(i == 0)
    def _():
        ssum[...] = jnp.zeros_like(ssum)
        ssq[...] = jnp.zeros_like(ssq)

    ssum[0:1, :] += jnp.sum(hm, axis=0, keepdims=True)
    ssq[0:1, :] += jnp.sum(hm * hm, axis=0, keepdims=True)

    @pl.when(i == NBLK - 1)
    def _():
        mu = ssum[0:1, :] / N
        var = ssq[0:1, :] / N - mu * mu
        st_ref[...] = jnp.concatenate(
            [mu, var, jnp.zeros((6, HID), jnp.float32)], axis=0)


def _k_comb(oparts, b):
    return pl.pallas_call(
        _comb_body,
        grid=(NBLK,),
        in_specs=[
            pl.BlockSpec((NSC, BN, C), lambda i: (0, i, 0)),
            pl.BlockSpec((NSC, BN, C), lambda i: (0, i, 0)),
            pl.BlockSpec((NSC, BN, C), lambda i: (0, i, 0)),
            pl.BlockSpec((NSC, BN, C), lambda i: (0, i, 0)),
            pl.BlockSpec((1, HID), lambda i: (0, 0)),
        ],
        out_specs=[
            pl.BlockSpec((BN, HID), lambda i: (i, 0)),
            pl.BlockSpec((8, HID), lambda i: (0, 0)),
        ],
        out_shape=[
            jax.ShapeDtypeStruct((N, HID), jnp.float32),
            jax.ShapeDtypeStruct((8, HID), jnp.float32),
        ],
        scratch_shapes=[
            pltpu.VMEM((8, HID), jnp.float32),
            pltpu.VMEM((8, HID), jnp.float32),
        ],
    )(*oparts, b.reshape(1, HID))


def _norm_body(h_ref, st_ref, g_ref, be_ref, o_ref):
    mu = st_ref[0:1, :]
    var = st_ref[1:2, :]
    xn = g_ref[...] * (h_ref[...] - mu) * lax.rsqrt(var + EPS_BN) + be_ref[...]
    o_ref[...] = jnp.maximum(xn, 0.0)


def _k_norm(hpre, stats, g, be):
    return pl.pallas_call(
        _norm_body,
        grid=(NBLK,),
        in_specs=[
            pl.BlockSpec((BN, HID), lambda i: (i, 0)),
            pl.BlockSpec((8, HID), lambda i: (0, 0)),
            pl.BlockSpec((1, HID), lambda i: (0, 0)),
            pl.BlockSpec((1, HID), lambda i: (0, 0)),
        ],
        out_specs=pl.BlockSpec((BN, HID), lambda i: (i, 0)),
        out_shape=jax.ShapeDtypeStruct((N, HID), jnp.float32),
    )(hpre, stats, g.reshape(1, HID), be.reshape(1, HID))


def _pool_body(h_ref, b_ref, bs_ref, wm_ref, wx_ref, bo_ref, o_ref,
               macc, cacc, xacc):
    i = pl.program_id(0)

    @pl.when(i == 0)
    def _():
        macc[...] = jnp.zeros_like(macc)
        cacc[...] = jnp.zeros_like(cacc)
        xacc[...] = jnp.full_like(xacc, -jnp.inf)

    hblk = h_ref[...]
    bl = b_ref[0]
    bs = bs_ref[0]
    valid_l = (i * BN + lax.broadcasted_iota(jnp.int32, (1, BN), 1)) < N
    valid_s = (i * BN + lax.broadcasted_iota(jnp.int32, (BN, 1), 0)) < N
    gid = lax.broadcasted_iota(jnp.int32, (G, BN), 0)
    onehot = jnp.where((bl == gid) & valid_l, 1.0, 0.0)
    macc[...] += jax.lax.dot_general(
        onehot, hblk, (((1,), (0,)), ((), ())),
        preferred_element_type=jnp.float32)
    cacc[...] += jax.lax.dot_general(
        onehot, jnp.ones((BN, HID), jnp.float32), (((1,), (0,)), ((), ())),
        preferred_element_type=jnp.float32)
    bmin = jnp.min(jnp.where(valid_l, bl, G))
    bmax = jnp.max(jnp.where(valid_l, bl, -1))
    for g in range(G):
        @pl.when((g >= bmin) & (g <= bmax))
        def _():
            sel = (bs == g) & valid_s
            mg = jnp.max(jnp.where(sel, hblk, -jnp.inf),
                         axis=0, keepdims=True)
            xacc[g:g + 1, :] = jnp.maximum(xacc[g:g + 1, :], mg)

    @pl.when(i == NBLK - 1)
    def _():
        gmean = macc[...] / jnp.maximum(cacc[...], 1.0)
        gmax = xacc[...]
        gmax = jnp.where(gmax == -jnp.inf, 0.0, gmax)
        o_ref[...] = (
            jax.lax.dot_general(gmean, wm_ref[...], (((1,), (0,)), ((), ())),
                                preferred_element_type=jnp.float32)
            + jax.lax.dot_general(gmax, wx_ref[...], (((1,), (0,)), ((), ())),
                                  preferred_element_type=jnp.float32)
            + bo_ref[...])


def _k_pool(hn, batch3d, batch_sub, Wout, bout):
    return pl.pallas_call(
        _pool_body,
        grid=(NBLK,),
        in_specs=[
            pl.BlockSpec((BN, HID), lambda i: (i, 0)),
            pl.BlockSpec((1, 1, BN), lambda i: (i, 0, 0)),
            pl.BlockSpec((1, BN, 1), lambda i: (i, 0, 0)),
            pl.BlockSpec((HID, HID), lambda i: (0, 0)),
            pl.BlockSpec((HID, HID), lambda i: (0, 0)),
            pl.BlockSpec((1, HID), lambda i: (0, 0)),
        ],
        out_specs=pl.BlockSpec((G, HID), lambda i: (0, 0)),
        out_shape=jax.ShapeDtypeStruct((G, HID), jnp.float32),
        scratch_shapes=[
            pltpu.VMEM((G, HID), jnp.float32),
            pltpu.VMEM((G, HID), jnp.float32),
            pltpu.VMEM((G, HID), jnp.float32),
        ],
    )(hn, batch3d, batch_sub, Wout[:HID], Wout[HID:], bout.reshape(1, HID))


def kernel(x, edge_index, batch, W1, a_src1, a_dst1, b1, g1, be1,
           W2, a_src2, a_dst2, b2, g2, be2, W3, a_src3, a_dst3, b3, g3, be3,
           Wout, bout):
    src = jnp.pad(edge_index[0], (0, EPAD - E)).reshape(ROWS, 128)
    dst = jnp.pad(edge_index[1], (0, EPAD - E)).reshape(ROWS, 128)
    zeros1 = jnp.zeros((3200,), jnp.float32)
    zeros_o = jnp.zeros((SL, C), jnp.float32)
    bpad = jnp.pad(batch, (0, NPAD - N), constant_values=G)
    batch3d = bpad.reshape(NBLK, 1, BN)
    batch_sub = bpad.reshape(NBLK, BN, 1)

    h = x
    for (W, a_s, a_d, b, g, be) in (
            (W1, a_src1, a_dst1, b1, g1, be1),
            (W2, a_src2, a_dst2, b2, g2, be2),
            (W3, a_src3, a_dst3, b3, g3, be3)):
        htabs, asv, adv = _k_embed(h, W, a_s, a_d)
        w, dpart = _sc_pass1(src, dst, asv.reshape(-1), adv.reshape(-1),
                             zeros1)
        denom = _k_dsum(dpart.reshape(NSC, H, N))
        alpha = _sc_alpha(dst, w, denom.reshape(-1))
        oparts = _sc_pass2(src, dst, alpha, htabs, zeros_o)
        hpre, stats = _k_comb(oparts, b)
        h = _k_norm(hpre, stats, g, be)

    return _k_pool(h, batch3d, batch_sub, Wout, bout)


# speedup vs baseline: 34.1904x; 1.1994x over previous
import dataclasses
import functools
import jax
import jax.numpy as jnp
from jax import lax
from jax.experimental import pallas as pl
from jax.experimental.pallas import tpu as pltpu
from jax.experimental.pallas import tpu_sc as plsc

N = 50000
E = 800000
H = 4
C = 32
HID = 128
G = 64
NEG_SLOPE = 0.2
EPS_BN = 1e-5

NSC = 2
NTI = 16
NW = NSC * NTI
EPT = 25600
EPAD = NW * EPT
ROWS = EPAD // 128
RPT = EPT // 128
CHUNK_ROWS = 8
NCHUNK = RPT // CHUNK_ROWS
SL = 3128
SL_LAST = N - 15 * SL

BN = 512
NBLK = (N + BN - 1) // BN
NPAD = NBLK * BN

_mesh = plsc.VectorSubcoreMesh(core_axis_name="c", subcore_axis_name="s")

_cp = pltpu.CompilerParams()
if "needs_layout_passes" in pltpu.CompilerParams.__dataclass_fields__:
    _cp = dataclasses.replace(_cp, needs_layout_passes=False)
if "use_tc_tiling_on_sc" in pltpu.CompilerParams.__dataclass_fields__:
    _cp = dataclasses.replace(_cp, use_tc_tiling_on_sc=False)


def _iota16():
    return lax.iota(jnp.int32, 16)


def _splat16(v):
    return lax.broadcast(v, (16,))


def _sc_pass1(src2d, dst2d, asv, adv, zeros1):
    def body(src_hbm, dst_hbm, as_hbm, ad_hbm, zeros_hbm, w_hbm, dpart_hbm,
             as_tab, ad_tab, src_buf, dst_buf, w_buf, dzero, dstage, dacc,
             sem):
        cid = lax.axis_index("c")
        sid = lax.axis_index("s")
        wid = cid * NTI + sid
        tr0 = wid * RPT

        pltpu.sync_copy(zeros_hbm.at[pl.ds(0, 3200)], dzero)

        for h in range(H):
            pltpu.sync_copy(as_hbm.at[pl.ds(h * N, N)], as_tab)
            pltpu.sync_copy(ad_hbm.at[pl.ds(h * N, N)], ad_tab)
            @pl.when(sid < NTI - 1)
            def _():
                pltpu.sync_copy(dzero.at[pl.ds(0, SL)],
                                dacc.at[pl.ds(sid * SL, SL)])
            @pl.when(sid == NTI - 1)
            def _():
                pltpu.sync_copy(dzero.at[pl.ds(0, SL_LAST)],
                                dacc.at[pl.ds(sid * SL, SL_LAST)])
            plsc.subcore_barrier()

            @pl.loop(0, NCHUNK)
            def _chunk(ch):
                r0 = tr0 + ch * CHUNK_ROWS
                pltpu.sync_copy(src_hbm.at[pl.ds(r0, CHUNK_ROWS)], src_buf)
                pltpu.sync_copy(dst_hbm.at[pl.ds(r0, CHUNK_ROWS)], dst_buf)

                @plsc.parallel_loop(0, CHUNK_ROWS * 8, unroll=4)
                def _grp(g):
                    row = g >> 3
                    off = (g & 7) * 16
                    sv = src_buf[row, pl.ds(off, 16)]
                    dv = dst_buf[row, pl.ds(off, 16)]
                    es = plsc.load_gather(as_tab, [sv])
                    ed = plsc.load_gather(ad_tab, [dv])
                    e = es + ed
                    e = jnp.where(e > 0, e, NEG_SLOPE * e)
                    wv = jnp.exp(e)
                    eid = _splat16(r0 * 128 + g * 16) + _iota16()
                    wv = jnp.where(eid < E, wv, 0.0)
                    w_buf[row, pl.ds(off, 16)] = wv

                pltpu.sync_copy(w_buf, w_hbm.at[h, pl.ds(r0, CHUNK_ROWS)])
                for j in range(CHUNK_ROWS):
                    pltpu.sync_copy(w_buf.at[j], dacc.at[dst_buf.at[j]],
                                    add=True)

            plsc.subcore_barrier()
            dbase = cid * (H * N) + h * N + sid * SL
            @pl.when(sid < NTI - 1)
            def _():
                pltpu.sync_copy(dacc.at[pl.ds(sid * SL, SL)],
                                dstage.at[pl.ds(0, SL)])
                pltpu.sync_copy(dstage.at[pl.ds(0, SL)],
                                dpart_hbm.at[pl.ds(dbase, SL)])
            @pl.when(sid == NTI - 1)
            def _():
                pltpu.sync_copy(dacc.at[pl.ds(sid * SL, SL_LAST)],
                                dstage.at[pl.ds(0, SL_LAST)])
                pltpu.sync_copy(dstage.at[pl.ds(0, SL_LAST)],
                                dpart_hbm.at[pl.ds(dbase, SL_LAST)])
            plsc.subcore_barrier()

    k = pl.kernel(
        body,
        out_type=[
            jax.ShapeDtypeStruct((H, ROWS, 128), jnp.float32),
            jax.ShapeDtypeStruct((NSC * H * N,), jnp.float32),
        ],
        mesh=_mesh,
        scratch_types=[
            pltpu.VMEM((N,), jnp.float32),
            pltpu.VMEM((N,), jnp.float32),
            pltpu.VMEM((CHUNK_ROWS, 128), jnp.int32),
            pltpu.VMEM((CHUNK_ROWS, 128), jnp.int32),
            pltpu.VMEM((CHUNK_ROWS, 128), jnp.float32),
            pltpu.VMEM((3200,), jnp.float32),
            pltpu.VMEM((SL,), jnp.float32),
            pltpu.VMEM_SHARED((N,), jnp.float32),
            pltpu.SemaphoreType.DMA,
        ],
        compiler_params=_cp,
    )
    return k(src2d, dst2d, asv, adv, zeros1)


def _sc_alpha(dst2d, w, denom):
    def body(dst_hbm, w_hbm, den_hbm, a_hbm,
             den_tab, dst_buf, w_buf, a_buf, sem):
        cid = lax.axis_index("c")
        sid = lax.axis_index("s")
        wid = cid * NTI + sid
        tr0 = wid * RPT

        for h in range(H):
            pltpu.sync_copy(den_hbm.at[pl.ds(h * N, N)], den_tab)

            @pl.loop(0, RPT // 8)
            def _chunk(ch):
                r0 = tr0 + ch * 8
                pltpu.sync_copy(dst_hbm.at[pl.ds(r0, 8)], dst_buf)
                pltpu.sync_copy(w_hbm.at[h, pl.ds(r0, 8)], w_buf)

                @plsc.parallel_loop(0, 64, unroll=4)
                def _grp(g):
                    row = g >> 3
                    off = (g & 7) * 16
                    dv_idx = dst_buf[row, pl.ds(off, 16)]
                    dvals = plsc.load_gather(den_tab, [dv_idx])
                    wv = w_buf[row, pl.ds(off, 16)]
                    a_buf[row, pl.ds(off, 16)] = wv / (dvals + 1e-16)

                pltpu.sync_copy(a_buf, a_hbm.at[h, pl.ds(r0, 8)])

    k = pl.kernel(
        body,
        out_type=jax.ShapeDtypeStruct((H, ROWS, 128), jnp.float32),
        mesh=_mesh,
        scratch_types=[
            pltpu.VMEM((N,), jnp.float32),
            pltpu.VMEM((8, 128), jnp.int32),
            pltpu.VMEM((8, 128), jnp.float32),
            pltpu.VMEM((8, 128), jnp.float32),
            pltpu.SemaphoreType.DMA,
        ],
        compiler_params=_cp,
    )
    return k(dst2d, w, denom)


P2R = 4
P2CHUNKS = RPT // P2R


def _sc_pass2(src2d, dst2d, alpha, htabs, zeros_o):
    def body(src_hbm, dst_hbm, a_hbm, h0, h1, h2, h3, zeros_hbm,
             op0, op1, op2, op3,
             src_buf, dst_buf, a_buf, rows3d, sbuf, oacc, sem):
        opart_list = [op0, op1, op2, op3]
        cid = lax.axis_index("c")
        sid = lax.axis_index("s")
        wid = cid * NTI + sid
        tr0 = wid * RPT
        htab_list = [h0, h1, h2, h3]

        for h in range(H):
            htab = htab_list[h]
            base = sid * SL
            @pl.when(sid < NTI - 1)
            def _():
                pltpu.sync_copy(zeros_hbm.at[pl.ds(0, SL)],
                                oacc.at[pl.ds(base, SL)])
            @pl.when(sid == NTI - 1)
            def _():
                pltpu.sync_copy(zeros_hbm.at[pl.ds(0, SL_LAST)],
                                oacc.at[pl.ds(base, SL_LAST)])
            plsc.subcore_barrier()

            @pl.loop(0, P2CHUNKS)
            def _chunk(ch):
                r0 = tr0 + ch * P2R
                pltpu.sync_copy(src_hbm.at[pl.ds(r0, P2R)], src_buf)
                pltpu.sync_copy(dst_hbm.at[pl.ds(r0, P2R)], dst_buf)
                pltpu.sync_copy(a_hbm.at[h, pl.ds(r0, P2R)], a_buf)

                cps = []
                for j in range(P2R):
                    cps.append(pltpu.async_copy(
                        htab.at[src_buf.at[j]], rows3d.at[j], sem))
                for cp in cps:
                    cp.wait()

                for j in range(P2R):
                    @plsc.parallel_loop(0, 128, unroll=4)
                    def _r(r):
                        sv = plsc.load_gather(a_buf,
                                              [_splat16(j), _splat16(r)])
                        rows3d[j, r, pl.ds(0, 16)] = (
                            rows3d[j, r, pl.ds(0, 16)] * sv)
                        rows3d[j, r, pl.ds(16, 16)] = (
                            rows3d[j, r, pl.ds(16, 16)] * sv)

                for j in range(P2R):
                    pltpu.sync_copy(rows3d.at[j],
                                    oacc.at[dst_buf.at[j]], add=True)

            plsc.subcore_barrier()
            opart_hbm = opart_list[h]
            for kk in range(35):
                pltpu.sync_copy(oacc.at[pl.ds(base + kk * 88, 88)], sbuf)
                pltpu.sync_copy(sbuf,
                                opart_hbm.at[cid, pl.ds(base + kk * 88, 88)])
            @pl.when(sid < NTI - 1)
            def _():
                t = SL - 35 * 88
                pltpu.sync_copy(oacc.at[pl.ds(base + 35 * 88, t)],
                                sbuf.at[pl.ds(0, t)])
                pltpu.sync_copy(sbuf.at[pl.ds(0, t)],
                                opart_hbm.at[cid, pl.ds(base + 35 * 88, t)])
            plsc.subcore_barrier()

    k = pl.kernel(
        body,
        out_type=[jax.ShapeDtypeStruct((NSC, N, C), jnp.float32)
                  for _ in range(H)],
        mesh=_mesh,
        scratch_types=[
            pltpu.VMEM((P2R, 128), jnp.int32),
            pltpu.VMEM((P2R, 128), jnp.int32),
            pltpu.VMEM((P2R, 128), jnp.float32),
            pltpu.VMEM((P2R, 128, C), jnp.float32),
            pltpu.VMEM((88, C), jnp.float32),
            pltpu.VMEM_SHARED((N, C), jnp.float32),
            pltpu.SemaphoreType.DMA,
        ],
        compiler_params=_cp,
    )
    return k(src2d, dst2d, alpha, *htabs, zeros_o)


def _embed_body(x_ref, w_ref, asf_ref, adf_ref,
                h0_ref, h1_ref, h2_ref, h3_ref, as_ref, ad_ref):
    hblk = jax.lax.dot_general(
        x_ref[...], w_ref[...], (((1,), (0,)), ((), ())),
        preferred_element_type=jnp.float32)
    for h, o in enumerate((h0_ref, h1_ref, h2_ref, h3_ref)):
        o[...] = hblk[:, h * C:(h + 1) * C]
    as_ref[...] = jax.lax.dot_general(
        asf_ref[...], hblk, (((1,), (1,)), ((), ())),
        preferred_element_type=jnp.float32)
    ad_ref[...] = jax.lax.dot_general(
        adf_ref[...], hblk, (((1,), (1,)), ((), ())),
        preferred_element_type=jnp.float32)


def _k_embed(hin, W, a_src, a_dst):
    din = hin.shape[1]
    eye4 = jnp.eye(H, dtype=jnp.float32)
    asf = (a_src[:, None, :] * eye4[:, :, None]).reshape(H, H * C)
    adf = (a_dst[:, None, :] * eye4[:, :, None]).reshape(H, H * C)
    outs = pl.pallas_call(
        _embed_body,
        grid=(NBLK,),
        in_specs=[
            pl.BlockSpec((BN, din), lambda i: (i, 0)),
            pl.BlockSpec((din, H * C), lambda i: (0, 0)),
            pl.BlockSpec((H, H * C), lambda i: (0, 0)),
            pl.BlockSpec((H, H * C), lambda i: (0, 0)),
        ],
        out_specs=[
            pl.BlockSpec((BN, C), lambda i: (i, 0)),
            pl.BlockSpec((BN, C), lambda i: (i, 0)),
            pl.BlockSpec((BN, C), lambda i: (i, 0)),
            pl.BlockSpec((BN, C), lambda i: (i, 0)),
            pl.BlockSpec((H, BN), lambda i: (0, i)),
            pl.BlockSpec((H, BN), lambda i: (0, i)),
        ],
        out_shape=[
            jax.ShapeDtypeStruct((N, C), jnp.float32),
            jax.ShapeDtypeStruct((N, C), jnp.float32),
            jax.ShapeDtypeStruct((N, C), jnp.float32),
            jax.ShapeDtypeStruct((N, C), jnp.float32),
            jax.ShapeDtypeStruct((H, N), jnp.float32),
            jax.ShapeDtypeStruct((H, N), jnp.float32),
        ],
    )(hin, W, asf, adf)
    return outs[:4], outs[4], outs[5]


def _dsum_body(d_ref, o_ref):
    o_ref[...] = d_ref[0] + d_ref[1]


def _k_dsum(dpart):
    return pl.pallas_call(
        _dsum_body,
        grid=(NBLK,),
        in_specs=[pl.BlockSpec((NSC, H, BN), lambda i: (0, 0, i))],
        out_specs=pl.BlockSpec((H, BN), lambda i: (0, i)),
        out_shape=jax.ShapeDtypeStruct((H, N), jnp.float32),
    )(dpart)


def _comb_body(p0_ref, p1_ref, p2_ref, p3_ref, b_ref, h_ref, st_ref,
               ssum, ssq):
    i = pl.program_id(0)
    hpre = jnp.concatenate(
        [p[0] + p[1] for p in (p0_ref, p1_ref, p2_ref, p3_ref)],
        axis=1) + b_ref[...]
    h_ref[...] = hpre
    valid = (i * BN + lax.broadcasted_iota(jnp.int32, (BN, 1), 0)) < N
    hm = jnp.where(valid, hpre, 0.0)

    @pl.when(i == 0)
    def _():
        ssum[...] = jnp.zeros_like(ssum)
        ssq[...] = jnp.zeros_like(ssq)

    ssum[0:1, :] += jnp.sum(hm, axis=0, keepdims=True)
    ssq[0:1, :] += jnp.sum(hm * hm, axis=0, keepdims=True)

    @pl.when(i == NBLK - 1)
    def _():
        mu = ssum[0:1, :] / N
        var = ssq[0:1, :] / N - mu * mu
        st_ref[...] = jnp.concatenate(
            [mu, var, jnp.zeros((6, HID), jnp.float32)], axis=0)


def _k_comb(oparts, b):
    return pl.pallas_call(
        _comb_body,
        grid=(NBLK,),
        in_specs=[
            pl.BlockSpec((NSC, BN, C), lambda i: (0, i, 0)),
            pl.BlockSpec((NSC, BN, C), lambda i: (0, i, 0)),
            pl.BlockSpec((NSC, BN, C), lambda i: (0, i, 0)),
            pl.BlockSpec((NSC, BN, C), lambda i: (0, i, 0)),
            pl.BlockSpec((1, HID), lambda i: (0, 0)),
        ],
        out_specs=[
            pl.BlockSpec((BN, HID), lambda i: (i, 0)),
            pl.BlockSpec((8, HID), lambda i: (0, 0)),
        ],
        out_shape=[
            jax.ShapeDtypeStruct((N, HID), jnp.float32),
            jax.ShapeDtypeStruct((8, HID), jnp.float32),
        ],
        scratch_shapes=[
            pltpu.VMEM((8, HID), jnp.float32),
            pltpu.VMEM((8, HID), jnp.float32),
        ],
    )(*oparts, b.reshape(1, HID))


def _norm_body(h_ref, st_ref, g_ref, be_ref, o_ref):
    mu = st_ref[0:1, :]
    var = st_ref[1:2, :]
    xn = g_ref[...] * (h_ref[...] - mu) * lax.rsqrt(var + EPS_BN) + be_ref[...]
    o_ref[...] = jnp.maximum(xn, 0.0)


def _k_norm(hpre, stats, g, be):
    return pl.pallas_call(
        _norm_body,
        grid=(NBLK,),
        in_specs=[
            pl.BlockSpec((BN, HID), lambda i: (i, 0)),
            pl.BlockSpec((8, HID), lambda i: (0, 0)),
            pl.BlockSpec((1, HID), lambda i: (0, 0)),
            pl.BlockSpec((1, HID), lambda i: (0, 0)),
        ],
        out_specs=pl.BlockSpec((BN, HID), lambda i: (i, 0)),
        out_shape=jax.ShapeDtypeStruct((N, HID), jnp.float32),
    )(hpre, stats, g.reshape(1, HID), be.reshape(1, HID))


def _pool_body(h_ref, b_ref, bs_ref, wm_ref, wx_ref, bo_ref, o_ref,
               macc, cacc, xacc):
    i = pl.program_id(0)

    @pl.when(i == 0)
    def _():
        macc[...] = jnp.zeros_like(macc)
        cacc[...] = jnp.zeros_like(cacc)
        xacc[...] = jnp.full_like(xacc, -jnp.inf)

    hblk = h_ref[...]
    bl = b_ref[0]
    bs = bs_ref[0]
    valid_l = (i * BN + lax.broadcasted_iota(jnp.int32, (1, BN), 1)) < N
    valid_s = (i * BN + lax.broadcasted_iota(jnp.int32, (BN, 1), 0)) < N
    gid = lax.broadcasted_iota(jnp.int32, (G, BN), 0)
    onehot = jnp.where((bl == gid) & valid_l, 1.0, 0.0)
    macc[...] += jax.lax.dot_general(
        onehot, hblk, (((1,), (0,)), ((), ())),
        preferred_element_type=jnp.float32)
    cacc[...] += jax.lax.dot_general(
        onehot, jnp.ones((BN, HID), jnp.float32), (((1,), (0,)), ((), ())),
        preferred_element_type=jnp.float32)
    bmin = jnp.min(jnp.where(valid_l, bl, G))
    bmax = jnp.max(jnp.where(valid_l, bl, -1))
    for g in range(G):
        @pl.when((g >= bmin) & (g <= bmax))
        def _():
            sel = (bs == g) & valid_s
            mg = jnp.max(jnp.where(sel, hblk, -jnp.inf),
                         axis=0, keepdims=True)
            xacc[g:g + 1, :] = jnp.maximum(xacc[g:g + 1, :], mg)

    @pl.when(i == NBLK - 1)
    def _():
        gmean = macc[...] / jnp.maximum(cacc[...], 1.0)
        gmax = xacc[...]
        gmax = jnp.where(gmax == -jnp.inf, 0.0, gmax)
        o_ref[...] = (
            jax.lax.dot_general(gmean, wm_ref[...], (((1,), (0,)), ((), ())),
                                preferred_element_type=jnp.float32)
            + jax.lax.dot_general(gmax, wx_ref[...], (((1,), (0,)), ((), ())),
                                  preferred_element_type=jnp.float32)
            + bo_ref[...])


def _k_pool(hn, batch3d, batch_sub, Wout, bout):
    return pl.pallas_call(
        _pool_body,
        grid=(NBLK,),
        in_specs=[
            pl.BlockSpec((BN, HID), lambda i: (i, 0)),
            pl.BlockSpec((1, 1, BN), lambda i: (i, 0, 0)),
            pl.BlockSpec((1, BN, 1), lambda i: (i, 0, 0)),
            pl.BlockSpec((HID, HID), lambda i: (0, 0)),
            pl.BlockSpec((HID, HID), lambda i: (0, 0)),
            pl.BlockSpec((1, HID), lambda i: (0, 0)),
        ],
        out_specs=pl.BlockSpec((G, HID), lambda i: (0, 0)),
        out_shape=jax.ShapeDtypeStruct((G, HID), jnp.float32),
        scratch_shapes=[
            pltpu.VMEM((G, HID), jnp.float32),
            pltpu.VMEM((G, HID), jnp.float32),
            pltpu.VMEM((G, HID), jnp.float32),
        ],
    )(hn, batch3d, batch_sub, Wout[:HID], Wout[HID:], bout.reshape(1, HID))


def kernel(x, edge_index, batch, W1, a_src1, a_dst1, b1, g1, be1,
           W2, a_src2, a_dst2, b2, g2, be2, W3, a_src3, a_dst3, b3, g3, be3,
           Wout, bout):
    src = jnp.pad(edge_index[0], (0, EPAD - E)).reshape(ROWS, 128)
    dst = jnp.pad(edge_index[1], (0, EPAD - E)).reshape(ROWS, 128)
    zeros1 = jnp.zeros((3200,), jnp.float32)
    zeros_o = jnp.zeros((SL, C), jnp.float32)
    bpad = jnp.pad(batch, (0, NPAD - N), constant_values=G)
    batch3d = bpad.reshape(NBLK, 1, BN)
    batch_sub = bpad.reshape(NBLK, BN, 1)

    h = x
    for (W, a_s, a_d, b, g, be) in (
            (W1, a_src1, a_dst1, b1, g1, be1),
            (W2, a_src2, a_dst2, b2, g2, be2),
            (W3, a_src3, a_dst3, b3, g3, be3)):
        htabs, asv, adv = _k_embed(h, W, a_s, a_d)
        w, dpart = _sc_pass1(src, dst, asv.reshape(-1), adv.reshape(-1),
                             zeros1)
        denom = _k_dsum(dpart.reshape(NSC, H, N))
        alpha = _sc_alpha(dst, w, denom.reshape(-1))
        oparts = _sc_pass2(src, dst, alpha, htabs, zeros_o)
        hpre, stats = _k_comb(oparts, b)
        h = _k_norm(hpre, stats, g, be)

    return _k_pool(h, batch3d, batch_sub, Wout, bout)


# speedup vs baseline: 34.6356x; 1.0130x over previous
import dataclasses
import functools
import jax
import jax.numpy as jnp
from jax import lax
from jax.experimental import pallas as pl
from jax.experimental.pallas import tpu as pltpu
from jax.experimental.pallas import tpu_sc as plsc

N = 50000
E = 800000
H = 4
C = 32
HID = 128
G = 64
NEG_SLOPE = 0.2
EPS_BN = 1e-5

NSC = 2
NTI = 16
NW = NSC * NTI
EPT = 25600
EPAD = NW * EPT
ROWS = EPAD // 128
RPT = EPT // 128
CHUNK_ROWS = 8
NCHUNK = RPT // CHUNK_ROWS
SL = 3128
SL_LAST = N - 15 * SL

BN = 512
NBLK = (N + BN - 1) // BN
NPAD = NBLK * BN

_mesh = plsc.VectorSubcoreMesh(core_axis_name="c", subcore_axis_name="s")

_cp = pltpu.CompilerParams()
if "needs_layout_passes" in pltpu.CompilerParams.__dataclass_fields__:
    _cp = dataclasses.replace(_cp, needs_layout_passes=False)
if "use_tc_tiling_on_sc" in pltpu.CompilerParams.__dataclass_fields__:
    _cp = dataclasses.replace(_cp, use_tc_tiling_on_sc=False)


def _iota16():
    return lax.iota(jnp.int32, 16)


def _splat16(v):
    return lax.broadcast(v, (16,))


def _sc_pass1(src2d, dst2d, asv, adv, zeros1):
    def body(src_hbm, dst_hbm, as_hbm, ad_hbm, zeros_hbm, w_hbm, dpart_hbm,
             as_tab, ad_tab, src_buf, dst_buf, w_buf, dzero, dstage, dacc,
             sem):
        cid = lax.axis_index("c")
        sid = lax.axis_index("s")
        wid = cid * NTI + sid
        tr0 = wid * RPT

        pltpu.sync_copy(zeros_hbm.at[pl.ds(0, 3200)], dzero)

        for h in range(H):
            pltpu.sync_copy(as_hbm.at[pl.ds(h * N, N)], as_tab)
            pltpu.sync_copy(ad_hbm.at[pl.ds(h * N, N)], ad_tab)
            @pl.when(sid < NTI - 1)
            def _():
                pltpu.sync_copy(dzero.at[pl.ds(0, SL)],
                                dacc.at[pl.ds(sid * SL, SL)])
            @pl.when(sid == NTI - 1)
            def _():
                pltpu.sync_copy(dzero.at[pl.ds(0, SL_LAST)],
                                dacc.at[pl.ds(sid * SL, SL_LAST)])
            plsc.subcore_barrier()

            @pl.loop(0, NCHUNK)
            def _chunk(ch):
                r0 = tr0 + ch * CHUNK_ROWS
                pltpu.sync_copy(src_hbm.at[pl.ds(r0, CHUNK_ROWS)], src_buf)
                pltpu.sync_copy(dst_hbm.at[pl.ds(r0, CHUNK_ROWS)], dst_buf)

                @plsc.parallel_loop(0, CHUNK_ROWS * 8, unroll=4)
                def _grp(g):
                    row = g >> 3
                    off = (g & 7) * 16
                    sv = src_buf[row, pl.ds(off, 16)]
                    dv = dst_buf[row, pl.ds(off, 16)]
                    es = plsc.load_gather(as_tab, [sv])
                    ed = plsc.load_gather(ad_tab, [dv])
                    e = es + ed
                    e = jnp.where(e > 0, e, NEG_SLOPE * e)
                    wv = jnp.exp(e)
                    eid = _splat16(r0 * 128 + g * 16) + _iota16()
                    wv = jnp.where(eid < E, wv, 0.0)
                    w_buf[row, pl.ds(off, 16)] = wv

                pltpu.sync_copy(w_buf, w_hbm.at[h, pl.ds(r0, CHUNK_ROWS)])
                for j in range(CHUNK_ROWS):
                    pltpu.sync_copy(w_buf.at[j], dacc.at[dst_buf.at[j]],
                                    add=True)

            plsc.subcore_barrier()
            dbase = cid * (H * N) + h * N + sid * SL
            @pl.when(sid < NTI - 1)
            def _():
                pltpu.sync_copy(dacc.at[pl.ds(sid * SL, SL)],
                                dstage.at[pl.ds(0, SL)])
                pltpu.sync_copy(dstage.at[pl.ds(0, SL)],
                                dpart_hbm.at[pl.ds(dbase, SL)])
            @pl.when(sid == NTI - 1)
            def _():
                pltpu.sync_copy(dacc.at[pl.ds(sid * SL, SL_LAST)],
                                dstage.at[pl.ds(0, SL_LAST)])
                pltpu.sync_copy(dstage.at[pl.ds(0, SL_LAST)],
                                dpart_hbm.at[pl.ds(dbase, SL_LAST)])
            plsc.subcore_barrier()

    k = pl.kernel(
        body,
        out_type=[
            jax.ShapeDtypeStruct((H, ROWS, 128), jnp.float32),
            jax.ShapeDtypeStruct((NSC * H * N,), jnp.float32),
        ],
        mesh=_mesh,
        scratch_types=[
            pltpu.VMEM((N,), jnp.float32),
            pltpu.VMEM((N,), jnp.float32),
            pltpu.VMEM((CHUNK_ROWS, 128), jnp.int32),
            pltpu.VMEM((CHUNK_ROWS, 128), jnp.int32),
            pltpu.VMEM((CHUNK_ROWS, 128), jnp.float32),
            pltpu.VMEM((3200,), jnp.float32),
            pltpu.VMEM((SL,), jnp.float32),
            pltpu.VMEM_SHARED((N,), jnp.float32),
            pltpu.SemaphoreType.DMA,
        ],
        compiler_params=_cp,
    )
    return k(src2d, dst2d, asv, adv, zeros1)


def _sc_alpha(dst2d, w, denom):
    def body(dst_hbm, w_hbm, den_hbm, a_hbm,
             den_tab, dst_buf, w_buf, a_buf, sem):
        cid = lax.axis_index("c")
        sid = lax.axis_index("s")
        wid = cid * NTI + sid
        tr0 = wid * RPT

        for h in range(H):
            pltpu.sync_copy(den_hbm.at[pl.ds(h * N, N)], den_tab)

            @pl.loop(0, RPT // 8)
            def _chunk(ch):
                r0 = tr0 + ch * 8
                pltpu.sync_copy(dst_hbm.at[pl.ds(r0, 8)], dst_buf)
                pltpu.sync_copy(w_hbm.at[h, pl.ds(r0, 8)], w_buf)

                @plsc.parallel_loop(0, 64, unroll=4)
                def _grp(g):
                    row = g >> 3
                    off = (g & 7) * 16
                    dv_idx = dst_buf[row, pl.ds(off, 16)]
                    dvals = plsc.load_gather(den_tab, [dv_idx])
                    wv = w_buf[row, pl.ds(off, 16)]
                    a_buf[row, pl.ds(off, 16)] = wv / (dvals + 1e-16)

                pltpu.sync_copy(a_buf, a_hbm.at[h, pl.ds(r0, 8)])

    k = pl.kernel(
        body,
        out_type=jax.ShapeDtypeStruct((H, ROWS, 128), jnp.float32),
        mesh=_mesh,
        scratch_types=[
            pltpu.VMEM((N,), jnp.float32),
            pltpu.VMEM((8, 128), jnp.int32),
            pltpu.VMEM((8, 128), jnp.float32),
            pltpu.VMEM((8, 128), jnp.float32),
            pltpu.SemaphoreType.DMA,
        ],
        compiler_params=_cp,
    )
    return k(dst2d, w, denom)


P2R = 4
P2CHUNKS = RPT // P2R


def _sc_pass2(src2d, dst2d, alpha, htabs, zeros_o):
    def body(src_hbm, dst_hbm, a_hbm, h0, h1, h2, h3, zeros_hbm,
             op0, op1, op2, op3,
             src_buf, dst_buf, a_buf, rows3d, sbuf, oacc, sem):
        opart_list = [op0, op1, op2, op3]
        cid = lax.axis_index("c")
        sid = lax.axis_index("s")
        wid = cid * NTI + sid
        tr0 = wid * RPT
        htab_list = [h0, h1, h2, h3]

        for h in range(H):
            htab = htab_list[h]
            base = sid * SL
            @pl.when(sid < NTI - 1)
            def _():
                pltpu.sync_copy(zeros_hbm.at[pl.ds(0, SL)],
                                oacc.at[pl.ds(base, SL)])
            @pl.when(sid == NTI - 1)
            def _():
                pltpu.sync_copy(zeros_hbm.at[pl.ds(0, SL_LAST)],
                                oacc.at[pl.ds(base, SL_LAST)])
            plsc.subcore_barrier()

            @pl.loop(0, P2CHUNKS)
            def _chunk(ch):
                r0 = tr0 + ch * P2R
                pltpu.sync_copy(src_hbm.at[pl.ds(r0, P2R)], src_buf)
                pltpu.sync_copy(dst_hbm.at[pl.ds(r0, P2R)], dst_buf)
                pltpu.sync_copy(a_hbm.at[h, pl.ds(r0, P2R)], a_buf)

                cps = []
                for j in range(P2R):
                    cps.append(pltpu.async_copy(
                        htab.at[src_buf.at[j]], rows3d.at[j], sem))
                for cp in cps:
                    cp.wait()

                for j in range(P2R):
                    @plsc.parallel_loop(0, 128, unroll=8)
                    def _r(r):
                        sv = plsc.load_gather(a_buf,
                                              [_splat16(j), _splat16(r)])
                        rows3d[j, r, pl.ds(0, 16)] = (
                            rows3d[j, r, pl.ds(0, 16)] * sv)
                        rows3d[j, r, pl.ds(16, 16)] = (
                            rows3d[j, r, pl.ds(16, 16)] * sv)

                scs = []
                for j in range(P2R):
                    scs.append(pltpu.async_copy(
                        rows3d.at[j], oacc.at[dst_buf.at[j]], sem,
                        add=True))
                for cp in scs:
                    cp.wait()

            plsc.subcore_barrier()
            opart_hbm = opart_list[h]
            for kk in range(35):
                pltpu.sync_copy(oacc.at[pl.ds(base + kk * 88, 88)], sbuf)
                pltpu.sync_copy(sbuf,
                                opart_hbm.at[cid, pl.ds(base + kk * 88, 88)])
            @pl.when(sid < NTI - 1)
            def _():
                t = SL - 35 * 88
                pltpu.sync_copy(oacc.at[pl.ds(base + 35 * 88, t)],
                                sbuf.at[pl.ds(0, t)])
                pltpu.sync_copy(sbuf.at[pl.ds(0, t)],
                                opart_hbm.at[cid, pl.ds(base + 35 * 88, t)])
            plsc.subcore_barrier()

    k = pl.kernel(
        body,
        out_type=[jax.ShapeDtypeStruct((NSC, N, C), jnp.float32)
                  for _ in range(H)],
        mesh=_mesh,
        scratch_types=[
            pltpu.VMEM((P2R, 128), jnp.int32),
            pltpu.VMEM((P2R, 128), jnp.int32),
            pltpu.VMEM((P2R, 128), jnp.float32),
            pltpu.VMEM((P2R, 128, C), jnp.float32),
            pltpu.VMEM((88, C), jnp.float32),
            pltpu.VMEM_SHARED((N, C), jnp.float32),
            pltpu.SemaphoreType.DMA,
        ],
        compiler_params=_cp,
    )
    return k(src2d, dst2d, alpha, *htabs, zeros_o)


def _embed_body(x_ref, w_ref, asf_ref, adf_ref,
                h0_ref, h1_ref, h2_ref, h3_ref, as_ref, ad_ref):
    hblk = jax.lax.dot_general(
        x_ref[...], w_ref[...], (((1,), (0,)), ((), ())),
        preferred_element_type=jnp.float32)
    for h, o in enumerate((h0_ref, h1_ref, h2_ref, h3_ref)):
        o[...] = hblk[:, h * C:(h + 1) * C]
    as_ref[...] = jax.lax.dot_general(
        asf_ref[...], hblk, (((1,), (1,)), ((), ())),
        preferred_element_type=jnp.float32)
    ad_ref[...] = jax.lax.dot_general(
        adf_ref[...], hblk, (((1,), (1,)), ((), ())),
        preferred_element_type=jnp.float32)


def _k_embed(hin, W, a_src, a_dst):
    din = hin.shape[1]
    eye4 = jnp.eye(H, dtype=jnp.float32)
    asf = (a_src[:, None, :] * eye4[:, :, None]).reshape(H, H * C)
    adf = (a_dst[:, None, :] * eye4[:, :, None]).reshape(H, H * C)
    outs = pl.pallas_call(
        _embed_body,
        grid=(NBLK,),
        in_specs=[
            pl.BlockSpec((BN, din), lambda i: (i, 0)),
            pl.BlockSpec((din, H * C), lambda i: (0, 0)),
            pl.BlockSpec((H, H * C), lambda i: (0, 0)),
            pl.BlockSpec((H, H * C), lambda i: (0, 0)),
        ],
        out_specs=[
            pl.BlockSpec((BN, C), lambda i: (i, 0)),
            pl.BlockSpec((BN, C), lambda i: (i, 0)),
            pl.BlockSpec((BN, C), lambda i: (i, 0)),
            pl.BlockSpec((BN, C), lambda i: (i, 0)),
            pl.BlockSpec((H, BN), lambda i: (0, i)),
            pl.BlockSpec((H, BN), lambda i: (0, i)),
        ],
        out_shape=[
            jax.ShapeDtypeStruct((N, C), jnp.float32),
            jax.ShapeDtypeStruct((N, C), jnp.float32),
            jax.ShapeDtypeStruct((N, C), jnp.float32),
            jax.ShapeDtypeStruct((N, C), jnp.float32),
            jax.ShapeDtypeStruct((H, N), jnp.float32),
            jax.ShapeDtypeStruct((H, N), jnp.float32),
        ],
    )(hin, W, asf, adf)
    return outs[:4], outs[4], outs[5]


def _dsum_body(d_ref, o_ref):
    o_ref[...] = d_ref[0] + d_ref[1]


def _k_dsum(dpart):
    return pl.pallas_call(
        _dsum_body,
        grid=(NBLK,),
        in_specs=[pl.BlockSpec((NSC, H, BN), lambda i: (0, 0, i))],
        out_specs=pl.BlockSpec((H, BN), lambda i: (0, i)),
        out_shape=jax.ShapeDtypeStruct((H, N), jnp.float32),
    )(dpart)


def _comb_body(p0_ref, p1_ref, p2_ref, p3_ref, b_ref, h_ref, st_ref,
               ssum, ssq):
    i = pl.program_id(0)
    hpre = jnp.concatenate(
        [p[0] + p[1] for p in (p0_ref, p1_ref, p2_ref, p3_ref)],
        axis=1) + b_ref[...]
    h_ref[...] = hpre
    valid = (i * BN + lax.broadcasted_iota(jnp.int32, (BN, 1), 0)) < N
    hm = jnp.where(valid, hpre, 0.0)

    @pl.when(i == 0)
    def _():
        ssum[...] = jnp.zeros_like(ssum)
        ssq[...] = jnp.zeros_like(ssq)

    ssum[0:1, :] += jnp.sum(hm, axis=0, keepdims=True)
    ssq[0:1, :] += jnp.sum(hm * hm, axis=0, keepdims=True)

    @pl.when(i == NBLK - 1)
    def _():
        mu = ssum[0:1, :] / N
        var = ssq[0:1, :] / N - mu * mu
        st_ref[...] = jnp.concatenate(
            [mu, var, jnp.zeros((6, HID), jnp.float32)], axis=0)


def _k_comb(oparts, b):
    return pl.pallas_call(
        _comb_body,
        grid=(NBLK,),
        in_specs=[
            pl.BlockSpec((NSC, BN, C), lambda i: (0, i, 0)),
            pl.BlockSpec((NSC, BN, C), lambda i: (0, i, 0)),
            pl.BlockSpec((NSC, BN, C), lambda i: (0, i, 0)),
            pl.BlockSpec((NSC, BN, C), lambda i: (0, i, 0)),
            pl.BlockSpec((1, HID), lambda i: (0, 0)),
        ],
        out_specs=[
            pl.BlockSpec((BN, HID), lambda i: (i, 0)),
            pl.BlockSpec((8, HID), lambda i: (0, 0)),
        ],
        out_shape=[
            jax.ShapeDtypeStruct((N, HID), jnp.float32),
            jax.ShapeDtypeStruct((8, HID), jnp.float32),
        ],
        scratch_shapes=[
            pltpu.VMEM((8, HID), jnp.float32),
            pltpu.VMEM((8, HID), jnp.float32),
        ],
    )(*oparts, b.reshape(1, HID))


def _norm_body(h_ref, st_ref, g_ref, be_ref, o_ref):
    mu = st_ref[0:1, :]
    var = st_ref[1:2, :]
    xn = g_ref[...] * (h_ref[...] - mu) * lax.rsqrt(var + EPS_BN) + be_ref[...]
    o_ref[...] = jnp.maximum(xn, 0.0)


def _k_norm(hpre, stats, g, be):
    return pl.pallas_call(
        _norm_body,
        grid=(NBLK,),
        in_specs=[
            pl.BlockSpec((BN, HID), lambda i: (i, 0)),
            pl.BlockSpec((8, HID), lambda i: (0, 0)),
            pl.BlockSpec((1, HID), lambda i: (0, 0)),
            pl.BlockSpec((1, HID), lambda i: (0, 0)),
        ],
        out_specs=pl.BlockSpec((BN, HID), lambda i: (i, 0)),
        out_shape=jax.ShapeDtypeStruct((N, HID), jnp.float32),
    )(hpre, stats, g.reshape(1, HID), be.reshape(1, HID))


def _pool_body(h_ref, b_ref, bs_ref, wm_ref, wx_ref, bo_ref, o_ref,
               macc, cacc, xacc):
    i = pl.program_id(0)

    @pl.when(i == 0)
    def _():
        macc[...] = jnp.zeros_like(macc)
        cacc[...] = jnp.zeros_like(cacc)
        xacc[...] = jnp.full_like(xacc, -jnp.inf)

    hblk = h_ref[...]
    bl = b_ref[0]
    bs = bs_ref[0]
    valid_l = (i * BN + lax.broadcasted_iota(jnp.int32, (1, BN), 1)) < N
    valid_s = (i * BN + lax.broadcasted_iota(jnp.int32, (BN, 1), 0)) < N
    gid = lax.broadcasted_iota(jnp.int32, (G, BN), 0)
    onehot = jnp.where((bl == gid) & valid_l, 1.0, 0.0)
    macc[...] += jax.lax.dot_general(
        onehot, hblk, (((1,), (0,)), ((), ())),
        preferred_element_type=jnp.float32)
    cacc[...] += jax.lax.dot_general(
        onehot, jnp.ones((BN, HID), jnp.float32), (((1,), (0,)), ((), ())),
        preferred_element_type=jnp.float32)
    bmin = jnp.min(jnp.where(valid_l, bl, G))
    bmax = jnp.max(jnp.where(valid_l, bl, -1))
    for g in range(G):
        @pl.when((g >= bmin) & (g <= bmax))
        def _():
            sel = (bs == g) & valid_s
            mg = jnp.max(jnp.where(sel, hblk, -jnp.inf),
                         axis=0, keepdims=True)
            xacc[g:g + 1, :] = jnp.maximum(xacc[g:g + 1, :], mg)

    @pl.when(i == NBLK - 1)
    def _():
        gmean = macc[...] / jnp.maximum(cacc[...], 1.0)
        gmax = xacc[...]
        gmax = jnp.where(gmax == -jnp.inf, 0.0, gmax)
        o_ref[...] = (
            jax.lax.dot_general(gmean, wm_ref[...], (((1,), (0,)), ((), ())),
                                preferred_element_type=jnp.float32)
            + jax.lax.dot_general(gmax, wx_ref[...], (((1,), (0,)), ((), ())),
                                  preferred_element_type=jnp.float32)
            + bo_ref[...])


def _k_pool(hn, batch3d, batch_sub, Wout, bout):
    return pl.pallas_call(
        _pool_body,
        grid=(NBLK,),
        in_specs=[
            pl.BlockSpec((BN, HID), lambda i: (i, 0)),
            pl.BlockSpec((1, 1, BN), lambda i: (i, 0, 0)),
            pl.BlockSpec((1, BN, 1), lambda i: (i, 0, 0)),
            pl.BlockSpec((HID, HID), lambda i: (0, 0)),
            pl.BlockSpec((HID, HID), lambda i: (0, 0)),
            pl.BlockSpec((1, HID), lambda i: (0, 0)),
        ],
        out_specs=pl.BlockSpec((G, HID), lambda i: (0, 0)),
        out_shape=jax.ShapeDtypeStruct((G, HID), jnp.float32),
        scratch_shapes=[
            pltpu.VMEM((G, HID), jnp.float32),
            pltpu.VMEM((G, HID), jnp.float32),
            pltpu.VMEM((G, HID), jnp.float32),
        ],
    )(hn, batch3d, batch_sub, Wout[:HID], Wout[HID:], bout.reshape(1, HID))


def kernel(x, edge_index, batch, W1, a_src1, a_dst1, b1, g1, be1,
           W2, a_src2, a_dst2, b2, g2, be2, W3, a_src3, a_dst3, b3, g3, be3,
           Wout, bout):
    src = jnp.pad(edge_index[0], (0, EPAD - E)).reshape(ROWS, 128)
    dst = jnp.pad(edge_index[1], (0, EPAD - E)).reshape(ROWS, 128)
    zeros1 = jnp.zeros((3200,), jnp.float32)
    zeros_o = jnp.zeros((SL, C), jnp.float32)
    bpad = jnp.pad(batch, (0, NPAD - N), constant_values=G)
    batch3d = bpad.reshape(NBLK, 1, BN)
    batch_sub = bpad.reshape(NBLK, BN, 1)

    h = x
    for (W, a_s, a_d, b, g, be) in (
            (W1, a_src1, a_dst1, b1, g1, be1),
            (W2, a_src2, a_dst2, b2, g2, be2),
            (W3, a_src3, a_dst3, b3, g3, be3)):
        htabs, asv, adv = _k_embed(h, W, a_s, a_d)
        w, dpart = _sc_pass1(src, dst, asv.reshape(-1), adv.reshape(-1),
                             zeros1)
        denom = _k_dsum(dpart.reshape(NSC, H, N))
        alpha = _sc_alpha(dst, w, denom.reshape(-1))
        oparts = _sc_pass2(src, dst, alpha, htabs, zeros_o)
        hpre, stats = _k_comb(oparts, b)
        h = _k_norm(hpre, stats, g, be)

    return _k_pool(h, batch3d, batch_sub, Wout, bout)


# speedup vs baseline: 41.1859x; 1.1891x over previous
import dataclasses
import functools
import jax
import jax.numpy as jnp
from jax import lax
from jax.experimental import pallas as pl
from jax.experimental.pallas import tpu as pltpu
from jax.experimental.pallas import tpu_sc as plsc

N = 50000
E = 800000
H = 4
C = 32
HID = 128
G = 64
NEG_SLOPE = 0.2
EPS_BN = 1e-5

NSC = 2
NTI = 16
NW = NSC * NTI
EPT = 25600
EPAD = NW * EPT
ROWS = EPAD // 128
RPT = EPT // 128
CHUNK_ROWS = 8
NCHUNK = RPT // CHUNK_ROWS
SL = 3128
SL_LAST = N - 15 * SL

BN = 512
NBLK = (N + BN - 1) // BN
NPAD = NBLK * BN

_mesh = plsc.VectorSubcoreMesh(core_axis_name="c", subcore_axis_name="s")

_cp = pltpu.CompilerParams()
if "needs_layout_passes" in pltpu.CompilerParams.__dataclass_fields__:
    _cp = dataclasses.replace(_cp, needs_layout_passes=False)
if "use_tc_tiling_on_sc" in pltpu.CompilerParams.__dataclass_fields__:
    _cp = dataclasses.replace(_cp, use_tc_tiling_on_sc=False)


def _iota16():
    return lax.iota(jnp.int32, 16)


def _splat16(v):
    return lax.broadcast(v, (16,))


def _sc_pass1(src2d, dst2d, asv, adv, zeros1):
    def body(src_hbm, dst_hbm, as_hbm, ad_hbm, zeros_hbm, w_hbm, dpart_hbm,
             as_tab, ad_tab, src_buf, dst_buf, w_buf, dzero, dstage, dacc,
             sem):
        cid = lax.axis_index("c")
        sid = lax.axis_index("s")
        wid = cid * NTI + sid
        tr0 = wid * RPT

        pltpu.sync_copy(zeros_hbm.at[pl.ds(0, 3200)], dzero)

        for h in range(H):
            pltpu.sync_copy(as_hbm.at[pl.ds(h * N, N)], as_tab)
            pltpu.sync_copy(ad_hbm.at[pl.ds(h * N, N)], ad_tab)
            @pl.when(sid < NTI - 1)
            def _():
                pltpu.sync_copy(dzero.at[pl.ds(0, SL)],
                                dacc.at[pl.ds(sid * SL, SL)])
            @pl.when(sid == NTI - 1)
            def _():
                pltpu.sync_copy(dzero.at[pl.ds(0, SL_LAST)],
                                dacc.at[pl.ds(sid * SL, SL_LAST)])
            plsc.subcore_barrier()

            @pl.loop(0, NCHUNK)
            def _chunk(ch):
                r0 = tr0 + ch * CHUNK_ROWS
                pltpu.sync_copy(src_hbm.at[pl.ds(r0, CHUNK_ROWS)], src_buf)
                pltpu.sync_copy(dst_hbm.at[pl.ds(r0, CHUNK_ROWS)], dst_buf)

                @plsc.parallel_loop(0, CHUNK_ROWS * 8, unroll=4)
                def _grp(g):
                    row = g >> 3
                    off = (g & 7) * 16
                    sv = src_buf[row, pl.ds(off, 16)]
                    dv = dst_buf[row, pl.ds(off, 16)]
                    es = plsc.load_gather(as_tab, [sv])
                    ed = plsc.load_gather(ad_tab, [dv])
                    e = es + ed
                    e = jnp.where(e > 0, e, NEG_SLOPE * e)
                    wv = jnp.exp(e)
                    eid = _splat16(r0 * 128 + g * 16) + _iota16()
                    wv = jnp.where(eid < E, wv, 0.0)
                    w_buf[row, pl.ds(off, 16)] = wv

                pltpu.sync_copy(w_buf, w_hbm.at[h, pl.ds(r0, CHUNK_ROWS)])
                for j in range(CHUNK_ROWS):
                    pltpu.sync_copy(w_buf.at[j], dacc.at[dst_buf.at[j]],
                                    add=True)

            plsc.subcore_barrier()
            dbase = cid * (H * N) + h * N + sid * SL
            @pl.when(sid < NTI - 1)
            def _():
                pltpu.sync_copy(dacc.at[pl.ds(sid * SL, SL)],
                                dstage.at[pl.ds(0, SL)])
                pltpu.sync_copy(dstage.at[pl.ds(0, SL)],
                                dpart_hbm.at[pl.ds(dbase, SL)])
            @pl.when(sid == NTI - 1)
            def _():
                pltpu.sync_copy(dacc.at[pl.ds(sid * SL, SL_LAST)],
                                dstage.at[pl.ds(0, SL_LAST)])
                pltpu.sync_copy(dstage.at[pl.ds(0, SL_LAST)],
                                dpart_hbm.at[pl.ds(dbase, SL_LAST)])
            plsc.subcore_barrier()

    k = pl.kernel(
        body,
        out_type=[
            jax.ShapeDtypeStruct((H, ROWS, 128), jnp.float32),
            jax.ShapeDtypeStruct((NSC * H * N,), jnp.float32),
        ],
        mesh=_mesh,
        scratch_types=[
            pltpu.VMEM((N,), jnp.float32),
            pltpu.VMEM((N,), jnp.float32),
            pltpu.VMEM((CHUNK_ROWS, 128), jnp.int32),
            pltpu.VMEM((CHUNK_ROWS, 128), jnp.int32),
            pltpu.VMEM((CHUNK_ROWS, 128), jnp.float32),
            pltpu.VMEM((3200,), jnp.float32),
            pltpu.VMEM((SL,), jnp.float32),
            pltpu.VMEM_SHARED((N,), jnp.float32),
            pltpu.SemaphoreType.DMA,
        ],
        compiler_params=_cp,
    )
    return k(src2d, dst2d, asv, adv, zeros1)


def _sc_alpha(dst2d, w, denom):
    def body(dst_hbm, w_hbm, den_hbm, a_hbm,
             den_tab, dst_buf, w_buf, a_buf, sem):
        cid = lax.axis_index("c")
        sid = lax.axis_index("s")
        wid = cid * NTI + sid
        tr0 = wid * RPT

        for h in range(H):
            pltpu.sync_copy(den_hbm.at[pl.ds(h * N, N)], den_tab)

            @pl.loop(0, RPT // 8)
            def _chunk(ch):
                r0 = tr0 + ch * 8
                pltpu.sync_copy(dst_hbm.at[pl.ds(r0, 8)], dst_buf)
                pltpu.sync_copy(w_hbm.at[h, pl.ds(r0, 8)], w_buf)

                @plsc.parallel_loop(0, 64, unroll=4)
                def _grp(g):
                    row = g >> 3
                    off = (g & 7) * 16
                    dv_idx = dst_buf[row, pl.ds(off, 16)]
                    dvals = plsc.load_gather(den_tab, [dv_idx])
                    wv = w_buf[row, pl.ds(off, 16)]
                    a_buf[row, pl.ds(off, 16)] = wv / (dvals + 1e-16)

                pltpu.sync_copy(a_buf, a_hbm.at[h, pl.ds(r0, 8)])

    k = pl.kernel(
        body,
        out_type=jax.ShapeDtypeStruct((H, ROWS, 128), jnp.float32),
        mesh=_mesh,
        scratch_types=[
            pltpu.VMEM((N,), jnp.float32),
            pltpu.VMEM((8, 128), jnp.int32),
            pltpu.VMEM((8, 128), jnp.float32),
            pltpu.VMEM((8, 128), jnp.float32),
            pltpu.SemaphoreType.DMA,
        ],
        compiler_params=_cp,
    )
    return k(dst2d, w, denom)


P2R = 2
P2CHUNKS = RPT // P2R


def _sc_pass2(src2d, dst2d, alpha, htabs, zeros_o):
    def body(src_hbm, dst_hbm, a_hbm, h0, h1, h2, h3, zeros_hbm,
             op0, op1, op2, op3,
             src0, dst0, a0, rows0, src1, dst1, a1, rows1, oacc,
             sem0, sem1):
        opart_list = [op0, op1, op2, op3]
        cid = lax.axis_index("c")
        sid = lax.axis_index("s")
        wid = cid * NTI + sid
        tr0 = wid * RPT
        htab_list = [h0, h1, h2, h3]
        sets = ((src0, dst0, a0, rows0, sem0), (src1, dst1, a1, rows1, sem1))

        for h in range(H):
            htab = htab_list[h]

            def load_idx(c, sbuf, dbuf, abuf):
                r0 = tr0 + c * P2R
                pltpu.sync_copy(src_hbm.at[pl.ds(r0, P2R)], sbuf)
                pltpu.sync_copy(dst_hbm.at[pl.ds(r0, P2R)], dbuf)
                pltpu.sync_copy(a_hbm.at[h, pl.ds(r0, P2R)], abuf)

            def issue_gathers(sbuf, rows, sem):
                for j in range(P2R):
                    pltpu.async_copy(htab.at[sbuf.at[j]], rows.at[j], sem)

            def wait_gathers(sbuf, rows, sem):
                for j in range(P2R):
                    pltpu.make_async_copy(htab.at[sbuf.at[j]], rows.at[j],
                                          sem).wait()

            def scale_and_scatter(dbuf, abuf, rows):
                for j in range(P2R):
                    @plsc.parallel_loop(0, 128, unroll=8)
                    def _r(r):
                        sv = plsc.load_gather(abuf,
                                              [_splat16(j), _splat16(r)])
                        rows[j, r, pl.ds(0, 16)] = (
                            rows[j, r, pl.ds(0, 16)] * sv)
                        rows[j, r, pl.ds(16, 16)] = (
                            rows[j, r, pl.ds(16, 16)] * sv)
                for j in range(P2R):
                    pltpu.sync_copy(rows.at[j], oacc.at[dbuf.at[j]],
                                    add=True)

            base = sid * SL
            @pl.when(sid < NTI - 1)
            def _():
                pltpu.sync_copy(zeros_hbm.at[pl.ds(0, SL)],
                                oacc.at[pl.ds(base, SL)])
            @pl.when(sid == NTI - 1)
            def _():
                pltpu.sync_copy(zeros_hbm.at[pl.ds(0, SL_LAST)],
                                oacc.at[pl.ds(base, SL_LAST)])
            plsc.subcore_barrier()

            load_idx(0, src0, dst0, a0)
            issue_gathers(src0, rows0, sem0)

            @pl.loop(0, P2CHUNKS // 2)
            def _it(it):
                c0 = 2 * it
                load_idx(c0 + 1, src1, dst1, a1)
                issue_gathers(src1, rows1, sem1)
                wait_gathers(src0, rows0, sem0)
                scale_and_scatter(dst0, a0, rows0)
                @pl.when(c0 + 2 < P2CHUNKS)
                def _():
                    load_idx(c0 + 2, src0, dst0, a0)
                    issue_gathers(src0, rows0, sem0)
                wait_gathers(src1, rows1, sem1)
                scale_and_scatter(dst1, a1, rows1)

            plsc.subcore_barrier()
            opart_hbm = opart_list[h]
            for kk in range(24):
                pltpu.sync_copy(oacc.at[pl.ds(base + kk * 128, 128)],
                                rows0.at[0])
                pltpu.sync_copy(rows0.at[0],
                                opart_hbm.at[cid, pl.ds(base + kk * 128, 128)])
            @pl.when(sid < NTI - 1)
            def _():
                t = SL - 24 * 128
                pltpu.sync_copy(oacc.at[pl.ds(base + 24 * 128, t)],
                                rows0.at[0, pl.ds(0, t)])
                pltpu.sync_copy(rows0.at[0, pl.ds(0, t)],
                                opart_hbm.at[cid, pl.ds(base + 24 * 128, t)])
            @pl.when(sid == NTI - 1)
            def _():
                t = SL_LAST - 24 * 128
                pltpu.sync_copy(oacc.at[pl.ds(base + 24 * 128, t)],
                                rows0.at[0, pl.ds(0, t)])
                pltpu.sync_copy(rows0.at[0, pl.ds(0, t)],
                                opart_hbm.at[cid, pl.ds(base + 24 * 128, t)])
            plsc.subcore_barrier()

    k = pl.kernel(
        body,
        out_type=[jax.ShapeDtypeStruct((NSC, N, C), jnp.float32)
                  for _ in range(H)],
        mesh=_mesh,
        scratch_types=[
            pltpu.VMEM((P2R, 128), jnp.int32),
            pltpu.VMEM((P2R, 128), jnp.int32),
            pltpu.VMEM((P2R, 128), jnp.float32),
            pltpu.VMEM((P2R, 128, C), jnp.float32),
            pltpu.VMEM((P2R, 128), jnp.int32),
            pltpu.VMEM((P2R, 128), jnp.int32),
            pltpu.VMEM((P2R, 128), jnp.float32),
            pltpu.VMEM((P2R, 128, C), jnp.float32),
            pltpu.VMEM_SHARED((N, C), jnp.float32),
            pltpu.SemaphoreType.DMA,
            pltpu.SemaphoreType.DMA,
        ],
        compiler_params=_cp,
    )
    return k(src2d, dst2d, alpha, *htabs, zeros_o)


def _embed_body(x_ref, w_ref, asf_ref, adf_ref,
                h0_ref, h1_ref, h2_ref, h3_ref, as_ref, ad_ref):
    hblk = jax.lax.dot_general(
        x_ref[...], w_ref[...], (((1,), (0,)), ((), ())),
        preferred_element_type=jnp.float32)
    for h, o in enumerate((h0_ref, h1_ref, h2_ref, h3_ref)):
        o[...] = hblk[:, h * C:(h + 1) * C]
    as_ref[...] = jax.lax.dot_general(
        asf_ref[...], hblk, (((1,), (1,)), ((), ())),
        preferred_element_type=jnp.float32)
    ad_ref[...] = jax.lax.dot_general(
        adf_ref[...], hblk, (((1,), (1,)), ((), ())),
        preferred_element_type=jnp.float32)


def _k_embed(hin, W, a_src, a_dst):
    din = hin.shape[1]
    eye4 = jnp.eye(H, dtype=jnp.float32)
    asf = (a_src[:, None, :] * eye4[:, :, None]).reshape(H, H * C)
    adf = (a_dst[:, None, :] * eye4[:, :, None]).reshape(H, H * C)
    outs = pl.pallas_call(
        _embed_body,
        grid=(NBLK,),
        in_specs=[
            pl.BlockSpec((BN, din), lambda i: (i, 0)),
            pl.BlockSpec((din, H * C), lambda i: (0, 0)),
            pl.BlockSpec((H, H * C), lambda i: (0, 0)),
            pl.BlockSpec((H, H * C), lambda i: (0, 0)),
        ],
        out_specs=[
            pl.BlockSpec((BN, C), lambda i: (i, 0)),
            pl.BlockSpec((BN, C), lambda i: (i, 0)),
            pl.BlockSpec((BN, C), lambda i: (i, 0)),
            pl.BlockSpec((BN, C), lambda i: (i, 0)),
            pl.BlockSpec((H, BN), lambda i: (0, i)),
            pl.BlockSpec((H, BN), lambda i: (0, i)),
        ],
        out_shape=[
            jax.ShapeDtypeStruct((N, C), jnp.float32),
            jax.ShapeDtypeStruct((N, C), jnp.float32),
            jax.ShapeDtypeStruct((N, C), jnp.float32),
            jax.ShapeDtypeStruct((N, C), jnp.float32),
            jax.ShapeDtypeStruct((H, N), jnp.float32),
            jax.ShapeDtypeStruct((H, N), jnp.float32),
        ],
    )(hin, W, asf, adf)
    return outs[:4], outs[4], outs[5]


def _dsum_body(d_ref, o_ref):
    o_ref[...] = d_ref[0] + d_ref[1]


def _k_dsum(dpart):
    return pl.pallas_call(
        _dsum_body,
        grid=(NBLK,),
        in_specs=[pl.BlockSpec((NSC, H, BN), lambda i: (0, 0, i))],
        out_specs=pl.BlockSpec((H, BN), lambda i: (0, i)),
        out_shape=jax.ShapeDtypeStruct((H, N), jnp.float32),
    )(dpart)


def _comb_body(p0_ref, p1_ref, p2_ref, p3_ref, b_ref, h_ref, st_ref,
               ssum, ssq):
    i = pl.program_id(0)
    hpre = jnp.concatenate(
        [p[0] + p[1] for p in (p0_ref, p1_ref, p2_ref, p3_ref)],
        axis=1) + b_ref[...]
    h_ref[...] = hpre
    valid = (i * BN + lax.broadcasted_iota(jnp.int32, (BN, 1), 0)) < N
    hm = jnp.where(valid, hpre, 0.0)

    @pl.when(i == 0)
    def _():
        ssum[...] = jnp.zeros_like(ssum)
        ssq[...] = jnp.zeros_like(ssq)

    ssum[0:1, :] += jnp.sum(hm, axis=0, keepdims=True)
    ssq[0:1, :] += jnp.sum(hm * hm, axis=0, keepdims=True)

    @pl.when(i == NBLK - 1)
    def _():
        mu = ssum[0:1, :] / N
        var = ssq[0:1, :] / N - mu * mu
        st_ref[...] = jnp.concatenate(
            [mu, var, jnp.zeros((6, HID), jnp.float32)], axis=0)


def _k_comb(oparts, b):
    return pl.pallas_call(
        _comb_body,
        grid=(NBLK,),
        in_specs=[
            pl.BlockSpec((NSC, BN, C), lambda i: (0, i, 0)),
            pl.BlockSpec((NSC, BN, C), lambda i: (0, i, 0)),
            pl.BlockSpec((NSC, BN, C), lambda i: (0, i, 0)),
            pl.BlockSpec((NSC, BN, C), lambda i: (0, i, 0)),
            pl.BlockSpec((1, HID), lambda i: (0, 0)),
        ],
        out_specs=[
            pl.BlockSpec((BN, HID), lambda i: (i, 0)),
            pl.BlockSpec((8, HID), lambda i: (0, 0)),
        ],
        out_shape=[
            jax.ShapeDtypeStruct((N, HID), jnp.float32),
            jax.ShapeDtypeStruct((8, HID), jnp.float32),
        ],
        scratch_shapes=[
            pltpu.VMEM((8, HID), jnp.float32),
            pltpu.VMEM((8, HID), jnp.float32),
        ],
    )(*oparts, b.reshape(1, HID))


def _norm_body(h_ref, st_ref, g_ref, be_ref, o_ref):
    mu = st_ref[0:1, :]
    var = st_ref[1:2, :]
    xn = g_ref[...] * (h_ref[...] - mu) * lax.rsqrt(var + EPS_BN) + be_ref[...]
    o_ref[...] = jnp.maximum(xn, 0.0)


def _k_norm(hpre, stats, g, be):
    return pl.pallas_call(
        _norm_body,
        grid=(NBLK,),
        in_specs=[
            pl.BlockSpec((BN, HID), lambda i: (i, 0)),
            pl.BlockSpec((8, HID), lambda i: (0, 0)),
            pl.BlockSpec((1, HID), lambda i: (0, 0)),
            pl.BlockSpec((1, HID), lambda i: (0, 0)),
        ],
        out_specs=pl.BlockSpec((BN, HID), lambda i: (i, 0)),
        out_shape=jax.ShapeDtypeStruct((N, HID), jnp.float32),
    )(hpre, stats, g.reshape(1, HID), be.reshape(1, HID))


def _pool_body(h_ref, b_ref, bs_ref, wm_ref, wx_ref, bo_ref, o_ref,
               macc, cacc, xacc):
    i = pl.program_id(0)

    @pl.when(i == 0)
    def _():
        macc[...] = jnp.zeros_like(macc)
        cacc[...] = jnp.zeros_like(cacc)
        xacc[...] = jnp.full_like(xacc, -jnp.inf)

    hblk = h_ref[...]
    bl = b_ref[0]
    bs = bs_ref[0]
    valid_l = (i * BN + lax.broadcasted_iota(jnp.int32, (1, BN), 1)) < N
    valid_s = (i * BN + lax.broadcasted_iota(jnp.int32, (BN, 1), 0)) < N
    gid = lax.broadcasted_iota(jnp.int32, (G, BN), 0)
    onehot = jnp.where((bl == gid) & valid_l, 1.0, 0.0)
    macc[...] += jax.lax.dot_general(
        onehot, hblk, (((1,), (0,)), ((), ())),
        preferred_element_type=jnp.float32)
    cacc[...] += jax.lax.dot_general(
        onehot, jnp.ones((BN, HID), jnp.float32), (((1,), (0,)), ((), ())),
        preferred_element_type=jnp.float32)
    bmin = jnp.min(jnp.where(valid_l, bl, G))
    bmax = jnp.max(jnp.where(valid_l, bl, -1))
    for g in range(G):
        @pl.when((g >= bmin) & (g <= bmax))
        def _():
            sel = (bs == g) & valid_s
            mg = jnp.max(jnp.where(sel, hblk, -jnp.inf),
                         axis=0, keepdims=True)
            xacc[g:g + 1, :] = jnp.maximum(xacc[g:g + 1, :], mg)

    @pl.when(i == NBLK - 1)
    def _():
        gmean = macc[...] / jnp.maximum(cacc[...], 1.0)
        gmax = xacc[...]
        gmax = jnp.where(gmax == -jnp.inf, 0.0, gmax)
        o_ref[...] = (
            jax.lax.dot_general(gmean, wm_ref[...], (((1,), (0,)), ((), ())),
                                preferred_element_type=jnp.float32)
            + jax.lax.dot_general(gmax, wx_ref[...], (((1,), (0,)), ((), ())),
                                  preferred_element_type=jnp.float32)
            + bo_ref[...])


def _k_pool(hn, batch3d, batch_sub, Wout, bout):
    return pl.pallas_call(
        _pool_body,
        grid=(NBLK,),
        in_specs=[
            pl.BlockSpec((BN, HID), lambda i: (i, 0)),
            pl.BlockSpec((1, 1, BN), lambda i: (i, 0, 0)),
            pl.BlockSpec((1, BN, 1), lambda i: (i, 0, 0)),
            pl.BlockSpec((HID, HID), lambda i: (0, 0)),
            pl.BlockSpec((HID, HID), lambda i: (0, 0)),
            pl.BlockSpec((1, HID), lambda i: (0, 0)),
        ],
        out_specs=pl.BlockSpec((G, HID), lambda i: (0, 0)),
        out_shape=jax.ShapeDtypeStruct((G, HID), jnp.float32),
        scratch_shapes=[
            pltpu.VMEM((G, HID), jnp.float32),
            pltpu.VMEM((G, HID), jnp.float32),
            pltpu.VMEM((G, HID), jnp.float32),
        ],
    )(hn, batch3d, batch_sub, Wout[:HID], Wout[HID:], bout.reshape(1, HID))


def kernel(x, edge_index, batch, W1, a_src1, a_dst1, b1, g1, be1,
           W2, a_src2, a_dst2, b2, g2, be2, W3, a_src3, a_dst3, b3, g3, be3,
           Wout, bout):
    src = jnp.pad(edge_index[0], (0, EPAD - E)).reshape(ROWS, 128)
    dst = jnp.pad(edge_index[1], (0, EPAD - E)).reshape(ROWS, 128)
    zeros1 = jnp.zeros((3200,), jnp.float32)
    zeros_o = jnp.zeros((SL, C), jnp.float32)
    bpad = jnp.pad(batch, (0, NPAD - N), constant_values=G)
    batch3d = bpad.reshape(NBLK, 1, BN)
    batch_sub = bpad.reshape(NBLK, BN, 1)

    h = x
    for (W, a_s, a_d, b, g, be) in (
            (W1, a_src1, a_dst1, b1, g1, be1),
            (W2, a_src2, a_dst2, b2, g2, be2),
            (W3, a_src3, a_dst3, b3, g3, be3)):
        htabs, asv, adv = _k_embed(h, W, a_s, a_d)
        w, dpart = _sc_pass1(src, dst, asv.reshape(-1), adv.reshape(-1),
                             zeros1)
        denom = _k_dsum(dpart.reshape(NSC, H, N))
        alpha = _sc_alpha(dst, w, denom.reshape(-1))
        oparts = _sc_pass2(src, dst, alpha, htabs, zeros_o)
        hpre, stats = _k_comb(oparts, b)
        h = _k_norm(hpre, stats, g, be)

    return _k_pool(h, batch3d, batch_sub, Wout, bout)


# speedup vs baseline: 43.1639x; 1.0480x over previous
import dataclasses
import functools
import jax
import jax.numpy as jnp
from jax import lax
from jax.experimental import pallas as pl
from jax.experimental.pallas import tpu as pltpu
from jax.experimental.pallas import tpu_sc as plsc

N = 50000
E = 800000
H = 4
C = 32
HID = 128
G = 64
NEG_SLOPE = 0.2
EPS_BN = 1e-5

NSC = 2
NTI = 16
NW = NSC * NTI
EPT = 25600
EPAD = NW * EPT
ROWS = EPAD // 128
RPT = EPT // 128
CHUNK_ROWS = 8
NCHUNK = RPT // CHUNK_ROWS
SL = 3128
SL_LAST = N - 15 * SL

BN = 512
NBLK = (N + BN - 1) // BN
NPAD = NBLK * BN

_mesh = plsc.VectorSubcoreMesh(core_axis_name="c", subcore_axis_name="s")

_cp = pltpu.CompilerParams()
if "needs_layout_passes" in pltpu.CompilerParams.__dataclass_fields__:
    _cp = dataclasses.replace(_cp, needs_layout_passes=False)
if "use_tc_tiling_on_sc" in pltpu.CompilerParams.__dataclass_fields__:
    _cp = dataclasses.replace(_cp, use_tc_tiling_on_sc=False)


def _iota16():
    return lax.iota(jnp.int32, 16)


def _splat16(v):
    return lax.broadcast(v, (16,))


def _sc_pass1(src2d, dst2d, asv, adv, zeros1):
    def body(src_hbm, dst_hbm, as_hbm, ad_hbm, zeros_hbm, w_hbm, dpart_hbm,
             as_tab, ad_tab, src_buf, dst_buf, w_buf, dzero, dstage, dacc,
             sem):
        cid = lax.axis_index("c")
        sid = lax.axis_index("s")
        wid = cid * NTI + sid
        tr0 = wid * RPT

        pltpu.sync_copy(zeros_hbm.at[pl.ds(0, 3200)], dzero)

        for h in range(H):
            pltpu.sync_copy(as_hbm.at[pl.ds(h * N, N)], as_tab)
            pltpu.sync_copy(ad_hbm.at[pl.ds(h * N, N)], ad_tab)
            @pl.when(sid < NTI - 1)
            def _():
                pltpu.sync_copy(dzero.at[pl.ds(0, SL)],
                                dacc.at[pl.ds(sid * SL, SL)])
            @pl.when(sid == NTI - 1)
            def _():
                pltpu.sync_copy(dzero.at[pl.ds(0, SL_LAST)],
                                dacc.at[pl.ds(sid * SL, SL_LAST)])
            plsc.subcore_barrier()

            @pl.loop(0, NCHUNK)
            def _chunk(ch):
                r0 = tr0 + ch * CHUNK_ROWS
                pltpu.sync_copy(src_hbm.at[pl.ds(r0, CHUNK_ROWS)], src_buf)
                pltpu.sync_copy(dst_hbm.at[pl.ds(r0, CHUNK_ROWS)], dst_buf)

                @plsc.parallel_loop(0, CHUNK_ROWS * 8, unroll=4)
                def _grp(g):
                    row = g >> 3
                    off = (g & 7) * 16
                    sv = src_buf[row, pl.ds(off, 16)]
                    dv = dst_buf[row, pl.ds(off, 16)]
                    es = plsc.load_gather(as_tab, [sv])
                    ed = plsc.load_gather(ad_tab, [dv])
                    e = es + ed
                    e = jnp.where(e > 0, e, NEG_SLOPE * e)
                    wv = jnp.exp(e)
                    eid = _splat16(r0 * 128 + g * 16) + _iota16()
                    wv = jnp.where(eid < E, wv, 0.0)
                    w_buf[row, pl.ds(off, 16)] = wv

                pltpu.sync_copy(w_buf, w_hbm.at[h, pl.ds(r0, CHUNK_ROWS)])
                for j in range(CHUNK_ROWS):
                    pltpu.sync_copy(w_buf.at[j], dacc.at[dst_buf.at[j]],
                                    add=True)

            plsc.subcore_barrier()
            dbase = cid * (H * N) + h * N + sid * SL
            @pl.when(sid < NTI - 1)
            def _():
                pltpu.sync_copy(dacc.at[pl.ds(sid * SL, SL)],
                                dstage.at[pl.ds(0, SL)])
                pltpu.sync_copy(dstage.at[pl.ds(0, SL)],
                                dpart_hbm.at[pl.ds(dbase, SL)])
            @pl.when(sid == NTI - 1)
            def _():
                pltpu.sync_copy(dacc.at[pl.ds(sid * SL, SL_LAST)],
                                dstage.at[pl.ds(0, SL_LAST)])
                pltpu.sync_copy(dstage.at[pl.ds(0, SL_LAST)],
                                dpart_hbm.at[pl.ds(dbase, SL_LAST)])
            plsc.subcore_barrier()

    k = pl.kernel(
        body,
        out_type=[
            jax.ShapeDtypeStruct((H, ROWS, 128), jnp.float32),
            jax.ShapeDtypeStruct((NSC * H * N,), jnp.float32),
        ],
        mesh=_mesh,
        scratch_types=[
            pltpu.VMEM((N,), jnp.float32),
            pltpu.VMEM((N,), jnp.float32),
            pltpu.VMEM((CHUNK_ROWS, 128), jnp.int32),
            pltpu.VMEM((CHUNK_ROWS, 128), jnp.int32),
            pltpu.VMEM((CHUNK_ROWS, 128), jnp.float32),
            pltpu.VMEM((3200,), jnp.float32),
            pltpu.VMEM((SL,), jnp.float32),
            pltpu.VMEM_SHARED((N,), jnp.float32),
            pltpu.SemaphoreType.DMA,
        ],
        compiler_params=_cp,
    )
    return k(src2d, dst2d, asv, adv, zeros1)


def _sc_alpha(dst2d, w, denom):
    def body(dst_hbm, w_hbm, den_hbm, a_hbm,
             den_tab, dst_buf, w_buf, a_buf, sem):
        cid = lax.axis_index("c")
        sid = lax.axis_index("s")
        wid = cid * NTI + sid
        tr0 = wid * RPT

        for h in range(H):
            pltpu.sync_copy(den_hbm.at[pl.ds(h * N, N)], den_tab)

            @pl.loop(0, RPT // 8)
            def _chunk(ch):
                r0 = tr0 + ch * 8
                pltpu.sync_copy(dst_hbm.at[pl.ds(r0, 8)], dst_buf)
                pltpu.sync_copy(w_hbm.at[h, pl.ds(r0, 8)], w_buf)

                @plsc.parallel_loop(0, 64, unroll=4)
                def _grp(g):
                    row = g >> 3
                    off = (g & 7) * 16
                    dv_idx = dst_buf[row, pl.ds(off, 16)]
                    dvals = plsc.load_gather(den_tab, [dv_idx])
                    wv = w_buf[row, pl.ds(off, 16)]
                    a_buf[row, pl.ds(off, 16)] = wv / (dvals + 1e-16)

                pltpu.sync_copy(a_buf, a_hbm.at[h, pl.ds(r0, 8)])

    k = pl.kernel(
        body,
        out_type=jax.ShapeDtypeStruct((H, ROWS, 128), jnp.float32),
        mesh=_mesh,
        scratch_types=[
            pltpu.VMEM((N,), jnp.float32),
            pltpu.VMEM((8, 128), jnp.int32),
            pltpu.VMEM((8, 128), jnp.float32),
            pltpu.VMEM((8, 128), jnp.float32),
            pltpu.SemaphoreType.DMA,
        ],
        compiler_params=_cp,
    )
    return k(dst2d, w, denom)


P2R = 2
P2CHUNKS = RPT // P2R


def _sc_pass2(src2d, dst2d, alpha, htabs, zeros_o):
    def body(src_hbm, dst_hbm, a_hbm, h0, h1, h2, h3, zeros_hbm,
             op0, op1, op2, op3,
             src0, dst0, a0, rows0, src1, dst1, a1, rows1, oacc,
             sem0, sem1, semi):
        opart_list = [op0, op1, op2, op3]
        cid = lax.axis_index("c")
        sid = lax.axis_index("s")
        wid = cid * NTI + sid
        tr0 = wid * RPT
        htab_list = [h0, h1, h2, h3]
        sets = ((src0, dst0, a0, rows0, sem0), (src1, dst1, a1, rows1, sem1))

        for h in range(H):
            htab = htab_list[h]

            def load_idx(c, sbuf, dbuf, abuf):
                r0 = tr0 + c * P2R
                cps = [
                    pltpu.async_copy(src_hbm.at[pl.ds(r0, P2R)], sbuf, semi),
                    pltpu.async_copy(dst_hbm.at[pl.ds(r0, P2R)], dbuf, semi),
                    pltpu.async_copy(a_hbm.at[h, pl.ds(r0, P2R)], abuf, semi),
                ]
                for cp in cps:
                    cp.wait()

            def issue_gathers(sbuf, rows, sem):
                for j in range(P2R):
                    pltpu.async_copy(htab.at[sbuf.at[j]], rows.at[j], sem)

            def wait_gathers(sbuf, rows, sem):
                for j in range(P2R):
                    pltpu.make_async_copy(htab.at[sbuf.at[j]], rows.at[j],
                                          sem).wait()

            def scale_and_scatter(dbuf, abuf, rows):
                for j in range(P2R):
                    @plsc.parallel_loop(0, 128, unroll=8)
                    def _r(r):
                        sv = plsc.load_gather(abuf,
                                              [_splat16(j), _splat16(r)])
                        rows[j, r, pl.ds(0, 16)] = (
                            rows[j, r, pl.ds(0, 16)] * sv)
                        rows[j, r, pl.ds(16, 16)] = (
                            rows[j, r, pl.ds(16, 16)] * sv)
                for j in range(P2R):
                    pltpu.sync_copy(rows.at[j], oacc.at[dbuf.at[j]],
                                    add=True)

            base = sid * SL
            @pl.when(sid < NTI - 1)
            def _():
                pltpu.sync_copy(zeros_hbm.at[pl.ds(0, SL)],
                                oacc.at[pl.ds(base, SL)])
            @pl.when(sid == NTI - 1)
            def _():
                pltpu.sync_copy(zeros_hbm.at[pl.ds(0, SL_LAST)],
                                oacc.at[pl.ds(base, SL_LAST)])
            plsc.subcore_barrier()

            load_idx(0, src0, dst0, a0)
            issue_gathers(src0, rows0, sem0)

            @pl.loop(0, P2CHUNKS // 2)
            def _it(it):
                c0 = 2 * it
                load_idx(c0 + 1, src1, dst1, a1)
                issue_gathers(src1, rows1, sem1)
                wait_gathers(src0, rows0, sem0)
                scale_and_scatter(dst0, a0, rows0)
                @pl.when(c0 + 2 < P2CHUNKS)
                def _():
                    load_idx(c0 + 2, src0, dst0, a0)
                    issue_gathers(src0, rows0, sem0)
                wait_gathers(src1, rows1, sem1)
                scale_and_scatter(dst1, a1, rows1)

            plsc.subcore_barrier()
            opart_hbm = opart_list[h]
            for kk in range(24):
                pltpu.sync_copy(oacc.at[pl.ds(base + kk * 128, 128)],
                                rows0.at[0])
                pltpu.sync_copy(rows0.at[0],
                                opart_hbm.at[cid, pl.ds(base + kk * 128, 128)])
            @pl.when(sid < NTI - 1)
            def _():
                t = SL - 24 * 128
                pltpu.sync_copy(oacc.at[pl.ds(base + 24 * 128, t)],
                                rows0.at[0, pl.ds(0, t)])
                pltpu.sync_copy(rows0.at[0, pl.ds(0, t)],
                                opart_hbm.at[cid, pl.ds(base + 24 * 128, t)])
            @pl.when(sid == NTI - 1)
            def _():
                t = SL_LAST - 24 * 128
                pltpu.sync_copy(oacc.at[pl.ds(base + 24 * 128, t)],
                                rows0.at[0, pl.ds(0, t)])
                pltpu.sync_copy(rows0.at[0, pl.ds(0, t)],
                                opart_hbm.at[cid, pl.ds(base + 24 * 128, t)])
            plsc.subcore_barrier()

    k = pl.kernel(
        body,
        out_type=[jax.ShapeDtypeStruct((NSC, N, C), jnp.float32)
                  for _ in range(H)],
        mesh=_mesh,
        scratch_types=[
            pltpu.VMEM((P2R, 128), jnp.int32),
            pltpu.VMEM((P2R, 128), jnp.int32),
            pltpu.VMEM((P2R, 128), jnp.float32),
            pltpu.VMEM((P2R, 128, C), jnp.float32),
            pltpu.VMEM((P2R, 128), jnp.int32),
            pltpu.VMEM((P2R, 128), jnp.int32),
            pltpu.VMEM((P2R, 128), jnp.float32),
            pltpu.VMEM((P2R, 128, C), jnp.float32),
            pltpu.VMEM_SHARED((N, C), jnp.float32),
            pltpu.SemaphoreType.DMA,
            pltpu.SemaphoreType.DMA,
            pltpu.SemaphoreType.DMA,
        ],
        compiler_params=_cp,
    )
    return k(src2d, dst2d, alpha, *htabs, zeros_o)


def _embed_body(x_ref, w_ref, asf_ref, adf_ref,
                h0_ref, h1_ref, h2_ref, h3_ref, as_ref, ad_ref):
    hblk = jax.lax.dot_general(
        x_ref[...], w_ref[...], (((1,), (0,)), ((), ())),
        preferred_element_type=jnp.float32)
    for h, o in enumerate((h0_ref, h1_ref, h2_ref, h3_ref)):
        o[...] = hblk[:, h * C:(h + 1) * C]
    as_ref[...] = jax.lax.dot_general(
        asf_ref[...], hblk, (((1,), (1,)), ((), ())),
        preferred_element_type=jnp.float32)
    ad_ref[...] = jax.lax.dot_general(
        adf_ref[...], hblk, (((1,), (1,)), ((), ())),
        preferred_element_type=jnp.float32)


def _k_embed(hin, W, a_src, a_dst):
    din = hin.shape[1]
    eye4 = jnp.eye(H, dtype=jnp.float32)
    asf = (a_src[:, None, :] * eye4[:, :, None]).reshape(H, H * C)
    adf = (a_dst[:, None, :] * eye4[:, :, None]).reshape(H, H * C)
    outs = pl.pallas_call(
        _embed_body,
        grid=(NBLK,),
        in_specs=[
            pl.BlockSpec((BN, din), lambda i: (i, 0)),
            pl.BlockSpec((din, H * C), lambda i: (0, 0)),
            pl.BlockSpec((H, H * C), lambda i: (0, 0)),
            pl.BlockSpec((H, H * C), lambda i: (0, 0)),
        ],
        out_specs=[
            pl.BlockSpec((BN, C), lambda i: (i, 0)),
            pl.BlockSpec((BN, C), lambda i: (i, 0)),
            pl.BlockSpec((BN, C), lambda i: (i, 0)),
            pl.BlockSpec((BN, C), lambda i: (i, 0)),
            pl.BlockSpec((H, BN), lambda i: (0, i)),
            pl.BlockSpec((H, BN), lambda i: (0, i)),
        ],
        out_shape=[
            jax.ShapeDtypeStruct((N, C), jnp.float32),
            jax.ShapeDtypeStruct((N, C), jnp.float32),
            jax.ShapeDtypeStruct((N, C), jnp.float32),
            jax.ShapeDtypeStruct((N, C), jnp.float32),
            jax.ShapeDtypeStruct((H, N), jnp.float32),
            jax.ShapeDtypeStruct((H, N), jnp.float32),
        ],
    )(hin, W, asf, adf)
    return outs[:4], outs[4], outs[5]


def _dsum_body(d_ref, o_ref):
    o_ref[...] = d_ref[0] + d_ref[1]


def _k_dsum(dpart):
    return pl.pallas_call(
        _dsum_body,
        grid=(NBLK,),
        in_specs=[pl.BlockSpec((NSC, H, BN), lambda i: (0, 0, i))],
        out_specs=pl.BlockSpec((H, BN), lambda i: (0, i)),
        out_shape=jax.ShapeDtypeStruct((H, N), jnp.float32),
    )(dpart)


def _comb_body(p0_ref, p1_ref, p2_ref, p3_ref, b_ref, h_ref, st_ref,
               ssum, ssq):
    i = pl.program_id(0)
    hpre = jnp.concatenate(
        [p[0] + p[1] for p in (p0_ref, p1_ref, p2_ref, p3_ref)],
        axis=1) + b_ref[...]
    h_ref[...] = hpre
    valid = (i * BN + lax.broadcasted_iota(jnp.int32, (BN, 1), 0)) < N
    hm = jnp.where(valid, hpre, 0.0)

    @pl.when(i == 0)
    def _():
        ssum[...] = jnp.zeros_like(ssum)
        ssq[...] = jnp.zeros_like(ssq)

    ssum[0:1, :] += jnp.sum(hm, axis=0, keepdims=True)
    ssq[0:1, :] += jnp.sum(hm * hm, axis=0, keepdims=True)

    @pl.when(i == NBLK - 1)
    def _():
        mu = ssum[0:1, :] / N
        var = ssq[0:1, :] / N - mu * mu
        st_ref[...] = jnp.concatenate(
            [mu, var, jnp.zeros((6, HID), jnp.float32)], axis=0)


def _k_comb(oparts, b):
    return pl.pallas_call(
        _comb_body,
        grid=(NBLK,),
        in_specs=[
            pl.BlockSpec((NSC, BN, C), lambda i: (0, i, 0)),
            pl.BlockSpec((NSC, BN, C), lambda i: (0, i, 0)),
            pl.BlockSpec((NSC, BN, C), lambda i: (0, i, 0)),
            pl.BlockSpec((NSC, BN, C), lambda i: (0, i, 0)),
            pl.BlockSpec((1, HID), lambda i: (0, 0)),
        ],
        out_specs=[
            pl.BlockSpec((BN, HID), lambda i: (i, 0)),
            pl.BlockSpec((8, HID), lambda i: (0, 0)),
        ],
        out_shape=[
            jax.ShapeDtypeStruct((N, HID), jnp.float32),
            jax.ShapeDtypeStruct((8, HID), jnp.float32),
        ],
        scratch_shapes=[
            pltpu.VMEM((8, HID), jnp.float32),
            pltpu.VMEM((8, HID), jnp.float32),
        ],
    )(*oparts, b.reshape(1, HID))


def _norm_body(h_ref, st_ref, g_ref, be_ref, o_ref):
    mu = st_ref[0:1, :]
    var = st_ref[1:2, :]
    xn = g_ref[...] * (h_ref[...] - mu) * lax.rsqrt(var + EPS_BN) + be_ref[...]
    o_ref[...] = jnp.maximum(xn, 0.0)


def _k_norm(hpre, stats, g, be):
    return pl.pallas_call(
        _norm_body,
        grid=(NBLK,),
        in_specs=[
            pl.BlockSpec((BN, HID), lambda i: (i, 0)),
            pl.BlockSpec((8, HID), lambda i: (0, 0)),
            pl.BlockSpec((1, HID), lambda i: (0, 0)),
            pl.BlockSpec((1, HID), lambda i: (0, 0)),
        ],
        out_specs=pl.BlockSpec((BN, HID), lambda i: (i, 0)),
        out_shape=jax.ShapeDtypeStruct((N, HID), jnp.float32),
    )(hpre, stats, g.reshape(1, HID), be.reshape(1, HID))


def _pool_body(h_ref, b_ref, bs_ref, wm_ref, wx_ref, bo_ref, o_ref,
               macc, cacc, xacc):
    i = pl.program_id(0)

    @pl.when(i == 0)
    def _():
        macc[...] = jnp.zeros_like(macc)
        cacc[...] = jnp.zeros_like(cacc)
        xacc[...] = jnp.full_like(xacc, -jnp.inf)

    hblk = h_ref[...]
    bl = b_ref[0]
    bs = bs_ref[0]
    valid_l = (i * BN + lax.broadcasted_iota(jnp.int32, (1, BN), 1)) < N
    valid_s = (i * BN + lax.broadcasted_iota(jnp.int32, (BN, 1), 0)) < N
    gid = lax.broadcasted_iota(jnp.int32, (G, BN), 0)
    onehot = jnp.where((bl == gid) & valid_l, 1.0, 0.0)
    macc[...] += jax.lax.dot_general(
        onehot, hblk, (((1,), (0,)), ((), ())),
        preferred_element_type=jnp.float32)
    cacc[...] += jax.lax.dot_general(
        onehot, jnp.ones((BN, HID), jnp.float32), (((1,), (0,)), ((), ())),
        preferred_element_type=jnp.float32)
    bmin = jnp.min(jnp.where(valid_l, bl, G))
    bmax = jnp.max(jnp.where(valid_l, bl, -1))
    for g in range(G):
        @pl.when((g >= bmin) & (g <= bmax))
        def _():
            sel = (bs == g) & valid_s
            mg = jnp.max(jnp.where(sel, hblk, -jnp.inf),
                         axis=0, keepdims=True)
            xacc[g:g + 1, :] = jnp.maximum(xacc[g:g + 1, :], mg)

    @pl.when(i == NBLK - 1)
    def _():
        gmean = macc[...] / jnp.maximum(cacc[...], 1.0)
        gmax = xacc[...]
        gmax = jnp.where(gmax == -jnp.inf, 0.0, gmax)
        o_ref[...] = (
            jax.lax.dot_general(gmean, wm_ref[...], (((1,), (0,)), ((), ())),
                                preferred_element_type=jnp.float32)
            + jax.lax.dot_general(gmax, wx_ref[...], (((1,), (0,)), ((), ())),
                                  preferred_element_type=jnp.float32)
            + bo_ref[...])


def _k_pool(hn, batch3d, batch_sub, Wout, bout):
    return pl.pallas_call(
        _pool_body,
        grid=(NBLK,),
        in_specs=[
            pl.BlockSpec((BN, HID), lambda i: (i, 0)),
            pl.BlockSpec((1, 1, BN), lambda i: (i, 0, 0)),
            pl.BlockSpec((1, BN, 1), lambda i: (i, 0, 0)),
            pl.BlockSpec((HID, HID), lambda i: (0, 0)),
            pl.BlockSpec((HID, HID), lambda i: (0, 0)),
            pl.BlockSpec((1, HID), lambda i: (0, 0)),
        ],
        out_specs=pl.BlockSpec((G, HID), lambda i: (0, 0)),
        out_shape=jax.ShapeDtypeStruct((G, HID), jnp.float32),
        scratch_shapes=[
            pltpu.VMEM((G, HID), jnp.float32),
            pltpu.VMEM((G, HID), jnp.float32),
            pltpu.VMEM((G, HID), jnp.float32),
        ],
    )(hn, batch3d, batch_sub, Wout[:HID], Wout[HID:], bout.reshape(1, HID))


def kernel(x, edge_index, batch, W1, a_src1, a_dst1, b1, g1, be1,
           W2, a_src2, a_dst2, b2, g2, be2, W3, a_src3, a_dst3, b3, g3, be3,
           Wout, bout):
    src = jnp.pad(edge_index[0], (0, EPAD - E)).reshape(ROWS, 128)
    dst = jnp.concatenate(
        [edge_index[1], jnp.arange(EPAD - E, dtype=jnp.int32) % N]
    ).reshape(ROWS, 128)
    zeros1 = jnp.zeros((3200,), jnp.float32)
    zeros_o = jnp.zeros((SL, C), jnp.float32)
    bpad = jnp.pad(batch, (0, NPAD - N), constant_values=G)
    batch3d = bpad.reshape(NBLK, 1, BN)
    batch_sub = bpad.reshape(NBLK, BN, 1)

    h = x
    for (W, a_s, a_d, b, g, be) in (
            (W1, a_src1, a_dst1, b1, g1, be1),
            (W2, a_src2, a_dst2, b2, g2, be2),
            (W3, a_src3, a_dst3, b3, g3, be3)):
        htabs, asv, adv = _k_embed(h, W, a_s, a_d)
        w, dpart = _sc_pass1(src, dst, asv.reshape(-1), adv.reshape(-1),
                             zeros1)
        denom = _k_dsum(dpart.reshape(NSC, H, N))
        alpha = _sc_alpha(dst, w, denom.reshape(-1))
        oparts = _sc_pass2(src, dst, alpha, htabs, zeros_o)
        hpre, stats = _k_comb(oparts, b)
        h = _k_norm(hpre, stats, g, be)

    return _k_pool(h, batch3d, batch_sub, Wout, bout)


# speedup vs baseline: 49.1674x; 1.1391x over previous
import dataclasses
import functools
import jax
import jax.numpy as jnp
from jax import lax
from jax.experimental import pallas as pl
from jax.experimental.pallas import tpu as pltpu
from jax.experimental.pallas import tpu_sc as plsc

N = 50000
E = 800000
H = 4
C = 32
HID = 128
G = 64
NEG_SLOPE = 0.2
EPS_BN = 1e-5

NSC = 2
NTI = 16
NW = NSC * NTI
EPT = 25600
EPAD = NW * EPT
ROWS = EPAD // 128
RPT = EPT // 128
CHUNK_ROWS = 8
NCHUNK = RPT // CHUNK_ROWS
SL = 3128
SL_LAST = N - 15 * SL

BN = 512
NBLK = (N + BN - 1) // BN
NPAD = NBLK * BN

_mesh = plsc.VectorSubcoreMesh(core_axis_name="c", subcore_axis_name="s")

_cp = pltpu.CompilerParams()
if "needs_layout_passes" in pltpu.CompilerParams.__dataclass_fields__:
    _cp = dataclasses.replace(_cp, needs_layout_passes=False)
if "use_tc_tiling_on_sc" in pltpu.CompilerParams.__dataclass_fields__:
    _cp = dataclasses.replace(_cp, use_tc_tiling_on_sc=False)


def _iota16():
    return lax.iota(jnp.int32, 16)


def _splat16(v):
    return lax.broadcast(v, (16,))


def _sc_pass1(src2d, dst2d, asv, adv, zeros1):
    def body(src_hbm, dst_hbm, as_hbm, ad_hbm, zeros_hbm, w_hbm, dpart_hbm,
             as_tab, ad_tab, src_buf, dst_buf, w_buf, dzero, dstage, dacc,
             sem):
        cid = lax.axis_index("c")
        sid = lax.axis_index("s")
        wid = cid * NTI + sid
        tr0 = wid * RPT

        pltpu.sync_copy(zeros_hbm.at[pl.ds(0, 3200)], dzero)

        for h in range(H):
            pltpu.sync_copy(as_hbm.at[pl.ds(h * N, N)], as_tab)
            pltpu.sync_copy(ad_hbm.at[pl.ds(h * N, N)], ad_tab)
            @pl.when(sid < NTI - 1)
            def _():
                pltpu.sync_copy(dzero.at[pl.ds(0, SL)],
                                dacc.at[pl.ds(sid * SL, SL)])
            @pl.when(sid == NTI - 1)
            def _():
                pltpu.sync_copy(dzero.at[pl.ds(0, SL_LAST)],
                                dacc.at[pl.ds(sid * SL, SL_LAST)])
            plsc.subcore_barrier()

            @pl.loop(0, NCHUNK)
            def _chunk(ch):
                r0 = tr0 + ch * CHUNK_ROWS
                pltpu.sync_copy(src_hbm.at[pl.ds(r0, CHUNK_ROWS)], src_buf)
                pltpu.sync_copy(dst_hbm.at[pl.ds(r0, CHUNK_ROWS)], dst_buf)

                @plsc.parallel_loop(0, CHUNK_ROWS * 8, unroll=4)
                def _grp(g):
                    row = g >> 3
                    off = (g & 7) * 16
                    sv = src_buf[row, pl.ds(off, 16)]
                    dv = dst_buf[row, pl.ds(off, 16)]
                    es = plsc.load_gather(as_tab, [sv])
                    ed = plsc.load_gather(ad_tab, [dv])
                    e = es + ed
                    e = jnp.where(e > 0, e, NEG_SLOPE * e)
                    wv = jnp.exp(e)
                    eid = _splat16(r0 * 128 + g * 16) + _iota16()
                    wv = jnp.where(eid < E, wv, 0.0)
                    w_buf[row, pl.ds(off, 16)] = wv

                pltpu.sync_copy(w_buf, w_hbm.at[h, pl.ds(r0, CHUNK_ROWS)])
                for j in range(CHUNK_ROWS):
                    pltpu.sync_copy(w_buf.at[j], dacc.at[dst_buf.at[j]],
                                    add=True)

            plsc.subcore_barrier()
            dbase = cid * (H * N) + h * N + sid * SL
            @pl.when(sid < NTI - 1)
            def _():
                pltpu.sync_copy(dacc.at[pl.ds(sid * SL, SL)],
                                dstage.at[pl.ds(0, SL)])
                pltpu.sync_copy(dstage.at[pl.ds(0, SL)],
                                dpart_hbm.at[pl.ds(dbase, SL)])
            @pl.when(sid == NTI - 1)
            def _():
                pltpu.sync_copy(dacc.at[pl.ds(sid * SL, SL_LAST)],
                                dstage.at[pl.ds(0, SL_LAST)])
                pltpu.sync_copy(dstage.at[pl.ds(0, SL_LAST)],
                                dpart_hbm.at[pl.ds(dbase, SL_LAST)])
            plsc.subcore_barrier()

    k = pl.kernel(
        body,
        out_type=[
            jax.ShapeDtypeStruct((H, ROWS, 128), jnp.float32),
            jax.ShapeDtypeStruct((NSC * H * N,), jnp.float32),
        ],
        mesh=_mesh,
        scratch_types=[
            pltpu.VMEM((N,), jnp.float32),
            pltpu.VMEM((N,), jnp.float32),
            pltpu.VMEM((CHUNK_ROWS, 128), jnp.int32),
            pltpu.VMEM((CHUNK_ROWS, 128), jnp.int32),
            pltpu.VMEM((CHUNK_ROWS, 128), jnp.float32),
            pltpu.VMEM((3200,), jnp.float32),
            pltpu.VMEM((SL,), jnp.float32),
            pltpu.VMEM_SHARED((N,), jnp.float32),
            pltpu.SemaphoreType.DMA,
        ],
        compiler_params=_cp,
    )
    return k(src2d, dst2d, asv, adv, zeros1)


def _sc_alpha(dst2d, w, denom):
    def body(dst_hbm, w_hbm, den_hbm, a_hbm,
             den_tab, dst_buf, w_buf, a_buf, sem):
        cid = lax.axis_index("c")
        sid = lax.axis_index("s")
        wid = cid * NTI + sid
        tr0 = wid * RPT

        for h in range(H):
            pltpu.sync_copy(den_hbm.at[pl.ds(h * N, N)], den_tab)

            @pl.loop(0, RPT // 8)
            def _chunk(ch):
                r0 = tr0 + ch * 8
                pltpu.sync_copy(dst_hbm.at[pl.ds(r0, 8)], dst_buf)
                pltpu.sync_copy(w_hbm.at[h, pl.ds(r0, 8)], w_buf)

                @plsc.parallel_loop(0, 64, unroll=4)
                def _grp(g):
                    row = g >> 3
                    off = (g & 7) * 16
                    dv_idx = dst_buf[row, pl.ds(off, 16)]
                    dvals = plsc.load_gather(den_tab, [dv_idx])
                    wv = w_buf[row, pl.ds(off, 16)]
                    a_buf[row, pl.ds(off, 16)] = wv / (dvals + 1e-16)

                pltpu.sync_copy(a_buf, a_hbm.at[h, pl.ds(r0, 8)])

    k = pl.kernel(
        body,
        out_type=jax.ShapeDtypeStruct((H, ROWS, 128), jnp.float32),
        mesh=_mesh,
        scratch_types=[
            pltpu.VMEM((N,), jnp.float32),
            pltpu.VMEM((8, 128), jnp.int32),
            pltpu.VMEM((8, 128), jnp.float32),
            pltpu.VMEM((8, 128), jnp.float32),
            pltpu.SemaphoreType.DMA,
        ],
        compiler_params=_cp,
    )
    return k(dst2d, w, denom)


P2R = 2
P2CHUNKS = RPT // P2R


def _sc_pass2(src2d, dst2d, alpha, htabs, zeros_o):
    def body(src_hbm, dst_hbm, a_hbm, h0, h1, h2, h3, zeros_hbm,
             op0, op1, op2, op3,
             src0, dst0, a0, rows0, src1, dst1, a1, rows1, oacc,
             sem0, sem1, semi):
        opart_list = [op0, op1, op2, op3]
        cid = lax.axis_index("c")
        sid = lax.axis_index("s")
        wid = cid * NTI + sid
        tr0 = wid * RPT
        htab_list = [h0, h1, h2, h3]
        sets = ((src0, dst0, a0, rows0, sem0), (src1, dst1, a1, rows1, sem1))

        for h in range(H):
            htab = htab_list[h]

            def load_idx(c, sbuf, dbuf, abuf):
                r0 = tr0 + c * P2R
                cps = [
                    pltpu.async_copy(src_hbm.at[pl.ds(r0, P2R)], sbuf, semi),
                    pltpu.async_copy(dst_hbm.at[pl.ds(r0, P2R)], dbuf, semi),
                    pltpu.async_copy(a_hbm.at[h, pl.ds(r0, P2R)], abuf, semi),
                ]
                for cp in cps:
                    cp.wait()

            def issue_gathers(sbuf, rows, sem):
                for j in range(P2R):
                    pltpu.async_copy(htab.at[sbuf.at[j]], rows.at[j], sem)

            def wait_gathers(sbuf, rows, sem):
                for j in range(P2R):
                    pltpu.make_async_copy(htab.at[sbuf.at[j]], rows.at[j],
                                          sem).wait()

            def scale_and_scatter(dbuf, abuf, rows):
                for j in range(P2R):
                    @plsc.parallel_loop(0, 128, unroll=8)
                    def _r(r):
                        sv = plsc.load_gather(abuf,
                                              [_splat16(j), _splat16(r)])
                        rows[j, r, pl.ds(0, 16)] = (
                            rows[j, r, pl.ds(0, 16)] * sv)
                        rows[j, r, pl.ds(16, 16)] = (
                            rows[j, r, pl.ds(16, 16)] * sv)
                for j in range(P2R):
                    pltpu.sync_copy(rows.at[j], oacc.at[dbuf.at[j]],
                                    add=True)

            base = sid * SL
            @pl.when(sid < NTI - 1)
            def _():
                pltpu.sync_copy(zeros_hbm.at[pl.ds(0, SL)],
                                oacc.at[pl.ds(base, SL)])
            @pl.when(sid == NTI - 1)
            def _():
                pltpu.sync_copy(zeros_hbm.at[pl.ds(0, SL_LAST)],
                                oacc.at[pl.ds(base, SL_LAST)])
            plsc.subcore_barrier()

            load_idx(0, src0, dst0, a0)
            issue_gathers(src0, rows0, sem0)

            @pl.loop(0, P2CHUNKS // 2)
            def _it(it):
                c0 = 2 * it
                load_idx(c0 + 1, src1, dst1, a1)
                issue_gathers(src1, rows1, sem1)
                wait_gathers(src0, rows0, sem0)
                scale_and_scatter(dst0, a0, rows0)
                @pl.when(c0 + 2 < P2CHUNKS)
                def _():
                    load_idx(c0 + 2, src0, dst0, a0)
                    issue_gathers(src0, rows0, sem0)
                wait_gathers(src1, rows1, sem1)
                scale_and_scatter(dst1, a1, rows1)

            plsc.subcore_barrier()
            opart_hbm = opart_list[h]
            for kk in range(24):
                pltpu.sync_copy(oacc.at[pl.ds(base + kk * 128, 128)],
                                rows0.at[0])
                pltpu.sync_copy(rows0.at[0],
                                opart_hbm.at[cid, pl.ds(base + kk * 128, 128)])
            @pl.when(sid < NTI - 1)
            def _():
                t = SL - 24 * 128
                pltpu.sync_copy(oacc.at[pl.ds(base + 24 * 128, t)],
                                rows0.at[0, pl.ds(0, t)])
                pltpu.sync_copy(rows0.at[0, pl.ds(0, t)],
                                opart_hbm.at[cid, pl.ds(base + 24 * 128, t)])
            @pl.when(sid == NTI - 1)
            def _():
                t = SL_LAST - 24 * 128
                pltpu.sync_copy(oacc.at[pl.ds(base + 24 * 128, t)],
                                rows0.at[0, pl.ds(0, t)])
                pltpu.sync_copy(rows0.at[0, pl.ds(0, t)],
                                opart_hbm.at[cid, pl.ds(base + 24 * 128, t)])
            plsc.subcore_barrier()

    k = pl.kernel(
        body,
        out_type=[jax.ShapeDtypeStruct((NSC, N, C), jnp.float32)
                  for _ in range(H)],
        mesh=_mesh,
        scratch_types=[
            pltpu.VMEM((P2R, 128), jnp.int32),
            pltpu.VMEM((P2R, 128), jnp.int32),
            pltpu.VMEM((P2R, 128), jnp.float32),
            pltpu.VMEM((P2R, 128, C), jnp.float32),
            pltpu.VMEM((P2R, 128), jnp.int32),
            pltpu.VMEM((P2R, 128), jnp.int32),
            pltpu.VMEM((P2R, 128), jnp.float32),
            pltpu.VMEM((P2R, 128, C), jnp.float32),
            pltpu.VMEM_SHARED((N, C), jnp.float32),
            pltpu.SemaphoreType.DMA,
            pltpu.SemaphoreType.DMA,
            pltpu.SemaphoreType.DMA,
        ],
        compiler_params=_cp,
    )
    return k(src2d, dst2d, alpha, *htabs, zeros_o)


def _embed_body(x_ref, w_ref, asf_ref, adf_ref,
                h0_ref, h1_ref, h2_ref, h3_ref, as_ref, ad_ref):
    hblk = jax.lax.dot_general(
        x_ref[...], w_ref[...], (((1,), (0,)), ((), ())),
        preferred_element_type=jnp.float32)
    for h, o in enumerate((h0_ref, h1_ref, h2_ref, h3_ref)):
        o[...] = hblk[:, h * C:(h + 1) * C]
    as_ref[...] = jax.lax.dot_general(
        asf_ref[...], hblk, (((1,), (1,)), ((), ())),
        preferred_element_type=jnp.float32)
    ad_ref[...] = jax.lax.dot_general(
        adf_ref[...], hblk, (((1,), (1,)), ((), ())),
        preferred_element_type=jnp.float32)


def _k_embed(hin, W, a_src, a_dst):
    din = hin.shape[1]
    eye4 = jnp.eye(H, dtype=jnp.float32)
    asf = (a_src[:, None, :] * eye4[:, :, None]).reshape(H, H * C)
    adf = (a_dst[:, None, :] * eye4[:, :, None]).reshape(H, H * C)
    outs = pl.pallas_call(
        _embed_body,
        grid=(NBLK,),
        in_specs=[
            pl.BlockSpec((BN, din), lambda i: (i, 0)),
            pl.BlockSpec((din, H * C), lambda i: (0, 0)),
            pl.BlockSpec((H, H * C), lambda i: (0, 0)),
            pl.BlockSpec((H, H * C), lambda i: (0, 0)),
        ],
        out_specs=[
            pl.BlockSpec((BN, C), lambda i: (i, 0)),
            pl.BlockSpec((BN, C), lambda i: (i, 0)),
            pl.BlockSpec((BN, C), lambda i: (i, 0)),
            pl.BlockSpec((BN, C), lambda i: (i, 0)),
            pl.BlockSpec((H, BN), lambda i: (0, i)),
            pl.BlockSpec((H, BN), lambda i: (0, i)),
        ],
        out_shape=[
            jax.ShapeDtypeStruct((N, C), jnp.float32),
            jax.ShapeDtypeStruct((N, C), jnp.float32),
            jax.ShapeDtypeStruct((N, C), jnp.float32),
            jax.ShapeDtypeStruct((N, C), jnp.float32),
            jax.ShapeDtypeStruct((H, N), jnp.float32),
            jax.ShapeDtypeStruct((H, N), jnp.float32),
        ],
    )(hin, W, asf, adf)
    return outs[:4], outs[4], outs[5]


def _dsum_body(d_ref, o_ref):
    o_ref[...] = d_ref[0] + d_ref[1]


def _k_dsum(dpart):
    return pl.pallas_call(
        _dsum_body,
        grid=(NBLK,),
        in_specs=[pl.BlockSpec((NSC, H, BN), lambda i: (0, 0, i))],
        out_specs=pl.BlockSpec((H, BN), lambda i: (0, i)),
        out_shape=jax.ShapeDtypeStruct((H, N), jnp.float32),
    )(dpart)


def _comb_body(p0_ref, p1_ref, p2_ref, p3_ref, d_ref, s_ref, b_ref,
               h_ref, st_ref, ssum, ssq):
    i = pl.program_id(0)
    dblk = d_ref[0] + d_ref[1]
    den_exp = jax.lax.dot_general(
        dblk, s_ref[...], (((0,), (0,)), ((), ())),
        preferred_element_type=jnp.float32) + 1e-16
    hpre = jnp.concatenate(
        [p[0] + p[1] for p in (p0_ref, p1_ref, p2_ref, p3_ref)],
        axis=1) / den_exp + b_ref[...]
    h_ref[...] = hpre
    valid = (i * BN + lax.broadcasted_iota(jnp.int32, (BN, 1), 0)) < N
    hm = jnp.where(valid, hpre, 0.0)

    @pl.when(i == 0)
    def _():
        ssum[...] = jnp.zeros_like(ssum)
        ssq[...] = jnp.zeros_like(ssq)

    ssum[0:1, :] += jnp.sum(hm, axis=0, keepdims=True)
    ssq[0:1, :] += jnp.sum(hm * hm, axis=0, keepdims=True)

    @pl.when(i == NBLK - 1)
    def _():
        mu = ssum[0:1, :] / N
        var = ssq[0:1, :] / N - mu * mu
        st_ref[...] = jnp.concatenate(
            [mu, var, jnp.zeros((6, HID), jnp.float32)], axis=0)


def _k_comb(oparts, dpart, b):
    sel = (jnp.eye(H, dtype=jnp.float32)[:, :, None]
           * jnp.ones((1, 1, C), jnp.float32)).reshape(H, HID)
    return pl.pallas_call(
        _comb_body,
        grid=(NBLK,),
        in_specs=[
            pl.BlockSpec((NSC, BN, C), lambda i: (0, i, 0)),
            pl.BlockSpec((NSC, BN, C), lambda i: (0, i, 0)),
            pl.BlockSpec((NSC, BN, C), lambda i: (0, i, 0)),
            pl.BlockSpec((NSC, BN, C), lambda i: (0, i, 0)),
            pl.BlockSpec((NSC, H, BN), lambda i: (0, 0, i)),
            pl.BlockSpec((H, HID), lambda i: (0, 0)),
            pl.BlockSpec((1, HID), lambda i: (0, 0)),
        ],
        out_specs=[
            pl.BlockSpec((BN, HID), lambda i: (i, 0)),
            pl.BlockSpec((8, HID), lambda i: (0, 0)),
        ],
        out_shape=[
            jax.ShapeDtypeStruct((N, HID), jnp.float32),
            jax.ShapeDtypeStruct((8, HID), jnp.float32),
        ],
        scratch_shapes=[
            pltpu.VMEM((8, HID), jnp.float32),
            pltpu.VMEM((8, HID), jnp.float32),
        ],
    )(*oparts, dpart, sel, b.reshape(1, HID))


def _norm_body(h_ref, st_ref, g_ref, be_ref, o_ref):
    mu = st_ref[0:1, :]
    var = st_ref[1:2, :]
    xn = g_ref[...] * (h_ref[...] - mu) * lax.rsqrt(var + EPS_BN) + be_ref[...]
    o_ref[...] = jnp.maximum(xn, 0.0)


def _k_norm(hpre, stats, g, be):
    return pl.pallas_call(
        _norm_body,
        grid=(NBLK,),
        in_specs=[
            pl.BlockSpec((BN, HID), lambda i: (i, 0)),
            pl.BlockSpec((8, HID), lambda i: (0, 0)),
            pl.BlockSpec((1, HID), lambda i: (0, 0)),
            pl.BlockSpec((1, HID), lambda i: (0, 0)),
        ],
        out_specs=pl.BlockSpec((BN, HID), lambda i: (i, 0)),
        out_shape=jax.ShapeDtypeStruct((N, HID), jnp.float32),
    )(hpre, stats, g.reshape(1, HID), be.reshape(1, HID))


def _pool_body(h_ref, b_ref, bs_ref, wm_ref, wx_ref, bo_ref, o_ref,
               macc, cacc, xacc):
    i = pl.program_id(0)

    @pl.when(i == 0)
    def _():
        macc[...] = jnp.zeros_like(macc)
        cacc[...] = jnp.zeros_like(cacc)
        xacc[...] = jnp.full_like(xacc, -jnp.inf)

    hblk = h_ref[...]
    bl = b_ref[0]
    bs = bs_ref[0]
    valid_l = (i * BN + lax.broadcasted_iota(jnp.int32, (1, BN), 1)) < N
    valid_s = (i * BN + lax.broadcasted_iota(jnp.int32, (BN, 1), 0)) < N
    gid = lax.broadcasted_iota(jnp.int32, (G, BN), 0)
    onehot = jnp.where((bl == gid) & valid_l, 1.0, 0.0)
    macc[...] += jax.lax.dot_general(
        onehot, hblk, (((1,), (0,)), ((), ())),
        preferred_element_type=jnp.float32)
    cacc[...] += jax.lax.dot_general(
        onehot, jnp.ones((BN, HID), jnp.float32), (((1,), (0,)), ((), ())),
        preferred_element_type=jnp.float32)
    bmin = jnp.min(jnp.where(valid_l, bl, G))
    bmax = jnp.max(jnp.where(valid_l, bl, -1))
    for g in range(G):
        @pl.when((g >= bmin) & (g <= bmax))
        def _():
            sel = (bs == g) & valid_s
            mg = jnp.max(jnp.where(sel, hblk, -jnp.inf),
                         axis=0, keepdims=True)
            xacc[g:g + 1, :] = jnp.maximum(xacc[g:g + 1, :], mg)

    @pl.when(i == NBLK - 1)
    def _():
        gmean = macc[...] / jnp.maximum(cacc[...], 1.0)
        gmax = xacc[...]
        gmax = jnp.where(gmax == -jnp.inf, 0.0, gmax)
        o_ref[...] = (
            jax.lax.dot_general(gmean, wm_ref[...], (((1,), (0,)), ((), ())),
                                preferred_element_type=jnp.float32)
            + jax.lax.dot_general(gmax, wx_ref[...], (((1,), (0,)), ((), ())),
                                  preferred_element_type=jnp.float32)
            + bo_ref[...])


def _k_pool(hn, batch3d, batch_sub, Wout, bout):
    return pl.pallas_call(
        _pool_body,
        grid=(NBLK,),
        in_specs=[
            pl.BlockSpec((BN, HID), lambda i: (i, 0)),
            pl.BlockSpec((1, 1, BN), lambda i: (i, 0, 0)),
            pl.BlockSpec((1, BN, 1), lambda i: (i, 0, 0)),
            pl.BlockSpec((HID, HID), lambda i: (0, 0)),
            pl.BlockSpec((HID, HID), lambda i: (0, 0)),
            pl.BlockSpec((1, HID), lambda i: (0, 0)),
        ],
        out_specs=pl.BlockSpec((G, HID), lambda i: (0, 0)),
        out_shape=jax.ShapeDtypeStruct((G, HID), jnp.float32),
        scratch_shapes=[
            pltpu.VMEM((G, HID), jnp.float32),
            pltpu.VMEM((G, HID), jnp.float32),
            pltpu.VMEM((G, HID), jnp.float32),
        ],
    )(hn, batch3d, batch_sub, Wout[:HID], Wout[HID:], bout.reshape(1, HID))


def kernel(x, edge_index, batch, W1, a_src1, a_dst1, b1, g1, be1,
           W2, a_src2, a_dst2, b2, g2, be2, W3, a_src3, a_dst3, b3, g3, be3,
           Wout, bout):
    src = jnp.pad(edge_index[0], (0, EPAD - E)).reshape(ROWS, 128)
    dst = jnp.concatenate(
        [edge_index[1], jnp.arange(EPAD - E, dtype=jnp.int32) % N]
    ).reshape(ROWS, 128)
    zeros1 = jnp.zeros((3200,), jnp.float32)
    zeros_o = jnp.zeros((SL, C), jnp.float32)
    bpad = jnp.pad(batch, (0, NPAD - N), constant_values=G)
    batch3d = bpad.reshape(NBLK, 1, BN)
    batch_sub = bpad.reshape(NBLK, BN, 1)

    h = x
    for (W, a_s, a_d, b, g, be) in (
            (W1, a_src1, a_dst1, b1, g1, be1),
            (W2, a_src2, a_dst2, b2, g2, be2),
            (W3, a_src3, a_dst3, b3, g3, be3)):
        htabs, asv, adv = _k_embed(h, W, a_s, a_d)
        w, dpart = _sc_pass1(src, dst, asv.reshape(-1), adv.reshape(-1),
                             zeros1)
        oparts = _sc_pass2(src, dst, w, htabs, zeros_o)
        hpre, stats = _k_comb(oparts, dpart.reshape(NSC, H, N), b)
        h = _k_norm(hpre, stats, g, be)

    return _k_pool(h, batch3d, batch_sub, Wout, bout)


# speedup vs baseline: 50.3104x; 1.0232x over previous
import dataclasses
import functools
import jax
import jax.numpy as jnp
from jax import lax
from jax.experimental import pallas as pl
from jax.experimental.pallas import tpu as pltpu
from jax.experimental.pallas import tpu_sc as plsc

N = 50000
E = 800000
H = 4
C = 32
HID = 128
G = 64
NEG_SLOPE = 0.2
EPS_BN = 1e-5

NSC = 2
NTI = 16
NW = NSC * NTI
EPT = 25600
EPAD = NW * EPT
ROWS = EPAD // 128
RPT = EPT // 128
CHUNK_ROWS = 8
NCHUNK = RPT // CHUNK_ROWS
SL = 3128
SL_LAST = N - 15 * SL

BN = 512
NBLK = (N + BN - 1) // BN
NPAD = NBLK * BN

_mesh = plsc.VectorSubcoreMesh(core_axis_name="c", subcore_axis_name="s")

_cp = pltpu.CompilerParams()
if "needs_layout_passes" in pltpu.CompilerParams.__dataclass_fields__:
    _cp = dataclasses.replace(_cp, needs_layout_passes=False)
if "use_tc_tiling_on_sc" in pltpu.CompilerParams.__dataclass_fields__:
    _cp = dataclasses.replace(_cp, use_tc_tiling_on_sc=False)


def _iota16():
    return lax.iota(jnp.int32, 16)


def _splat16(v):
    return lax.broadcast(v, (16,))


def _sc_pass1(src2d, dst2d, asv, adv, zeros1):
    def body(src_hbm, dst_hbm, as_hbm, ad_hbm, zeros_hbm, w_hbm, dpart_hbm,
             as_tab, ad_tab, src_buf, dst_buf, w_buf, dzero, dstage, dacc,
             sem):
        cid = lax.axis_index("c")
        sid = lax.axis_index("s")
        wid = cid * NTI + sid
        tr0 = wid * RPT

        pltpu.sync_copy(zeros_hbm.at[pl.ds(0, 3200)], dzero)

        for h in range(H):
            pltpu.sync_copy(as_hbm.at[pl.ds(h * N, N)], as_tab)
            pltpu.sync_copy(ad_hbm.at[pl.ds(h * N, N)], ad_tab)
            @pl.when(sid < NTI - 1)
            def _():
                pltpu.sync_copy(dzero.at[pl.ds(0, SL)],
                                dacc.at[pl.ds(sid * SL, SL)])
            @pl.when(sid == NTI - 1)
            def _():
                pltpu.sync_copy(dzero.at[pl.ds(0, SL_LAST)],
                                dacc.at[pl.ds(sid * SL, SL_LAST)])
            plsc.subcore_barrier()

            @pl.loop(0, NCHUNK)
            def _chunk(ch):
                r0 = tr0 + ch * CHUNK_ROWS
                cps = [
                    pltpu.async_copy(src_hbm.at[pl.ds(r0, CHUNK_ROWS)],
                                     src_buf, sem),
                    pltpu.async_copy(dst_hbm.at[pl.ds(r0, CHUNK_ROWS)],
                                     dst_buf, sem),
                ]
                for cp in cps:
                    cp.wait()

                @plsc.parallel_loop(0, CHUNK_ROWS * 8, unroll=4)
                def _grp(g):
                    row = g >> 3
                    off = (g & 7) * 16
                    sv = src_buf[row, pl.ds(off, 16)]
                    dv = dst_buf[row, pl.ds(off, 16)]
                    es = plsc.load_gather(as_tab, [sv])
                    ed = plsc.load_gather(ad_tab, [dv])
                    e = es + ed
                    e = jnp.where(e > 0, e, NEG_SLOPE * e)
                    wv = jnp.exp(e)
                    eid = _splat16(r0 * 128 + g * 16) + _iota16()
                    wv = jnp.where(eid < E, wv, 0.0)
                    w_buf[row, pl.ds(off, 16)] = wv

                pltpu.sync_copy(w_buf, w_hbm.at[h, pl.ds(r0, CHUNK_ROWS)])
                for j in range(CHUNK_ROWS):
                    pltpu.sync_copy(w_buf.at[j], dacc.at[dst_buf.at[j]],
                                    add=True)

            plsc.subcore_barrier()
            dbase = cid * (H * N) + h * N + sid * SL
            @pl.when(sid < NTI - 1)
            def _():
                pltpu.sync_copy(dacc.at[pl.ds(sid * SL, SL)],
                                dstage.at[pl.ds(0, SL)])
                pltpu.sync_copy(dstage.at[pl.ds(0, SL)],
                                dpart_hbm.at[pl.ds(dbase, SL)])
            @pl.when(sid == NTI - 1)
            def _():
                pltpu.sync_copy(dacc.at[pl.ds(sid * SL, SL_LAST)],
                                dstage.at[pl.ds(0, SL_LAST)])
                pltpu.sync_copy(dstage.at[pl.ds(0, SL_LAST)],
                                dpart_hbm.at[pl.ds(dbase, SL_LAST)])
            plsc.subcore_barrier()

    k = pl.kernel(
        body,
        out_type=[
            jax.ShapeDtypeStruct((H, ROWS, 128), jnp.float32),
            jax.ShapeDtypeStruct((NSC * H * N,), jnp.float32),
        ],
        mesh=_mesh,
        scratch_types=[
            pltpu.VMEM((N,), jnp.float32),
            pltpu.VMEM((N,), jnp.float32),
            pltpu.VMEM((CHUNK_ROWS, 128), jnp.int32),
            pltpu.VMEM((CHUNK_ROWS, 128), jnp.int32),
            pltpu.VMEM((CHUNK_ROWS, 128), jnp.float32),
            pltpu.VMEM((3200,), jnp.float32),
            pltpu.VMEM((SL,), jnp.float32),
            pltpu.VMEM_SHARED((N,), jnp.float32),
            pltpu.SemaphoreType.DMA,
        ],
        compiler_params=_cp,
    )
    return k(src2d, dst2d, asv, adv, zeros1)


def _sc_alpha(dst2d, w, denom):
    def body(dst_hbm, w_hbm, den_hbm, a_hbm,
             den_tab, dst_buf, w_buf, a_buf, sem):
        cid = lax.axis_index("c")
        sid = lax.axis_index("s")
        wid = cid * NTI + sid
        tr0 = wid * RPT

        for h in range(H):
            pltpu.sync_copy(den_hbm.at[pl.ds(h * N, N)], den_tab)

            @pl.loop(0, RPT // 8)
            def _chunk(ch):
                r0 = tr0 + ch * 8
                pltpu.sync_copy(dst_hbm.at[pl.ds(r0, 8)], dst_buf)
                pltpu.sync_copy(w_hbm.at[h, pl.ds(r0, 8)], w_buf)

                @plsc.parallel_loop(0, 64, unroll=4)
                def _grp(g):
                    row = g >> 3
                    off = (g & 7) * 16
                    dv_idx = dst_buf[row, pl.ds(off, 16)]
                    dvals = plsc.load_gather(den_tab, [dv_idx])
                    wv = w_buf[row, pl.ds(off, 16)]
                    a_buf[row, pl.ds(off, 16)] = wv / (dvals + 1e-16)

                pltpu.sync_copy(a_buf, a_hbm.at[h, pl.ds(r0, 8)])

    k = pl.kernel(
        body,
        out_type=jax.ShapeDtypeStruct((H, ROWS, 128), jnp.float32),
        mesh=_mesh,
        scratch_types=[
            pltpu.VMEM((N,), jnp.float32),
            pltpu.VMEM((8, 128), jnp.int32),
            pltpu.VMEM((8, 128), jnp.float32),
            pltpu.VMEM((8, 128), jnp.float32),
            pltpu.SemaphoreType.DMA,
        ],
        compiler_params=_cp,
    )
    return k(dst2d, w, denom)


P2R = 2
P2CHUNKS = RPT // P2R


def _sc_pass2(src2d, dst2d, alpha, htabs, zeros_o):
    def body(src_hbm, dst_hbm, a_hbm, h0, h1, h2, h3, zeros_hbm,
             op0, op1, op2, op3,
             src0, dst0, a0, rows0, src1, dst1, a1, rows1, oacc,
             sem0, sem1, semi):
        opart_list = [op0, op1, op2, op3]
        cid = lax.axis_index("c")
        sid = lax.axis_index("s")
        wid = cid * NTI + sid
        tr0 = wid * RPT
        htab_list = [h0, h1, h2, h3]
        sets = ((src0, dst0, a0, rows0, sem0), (src1, dst1, a1, rows1, sem1))

        for h in range(H):
            htab = htab_list[h]

            def load_idx(c, sbuf, dbuf, abuf):
                r0 = tr0 + c * P2R
                cps = [
                    pltpu.async_copy(src_hbm.at[pl.ds(r0, P2R)], sbuf, semi),
                    pltpu.async_copy(dst_hbm.at[pl.ds(r0, P2R)], dbuf, semi),
                    pltpu.async_copy(a_hbm.at[h, pl.ds(r0, P2R)], abuf, semi),
                ]
                for cp in cps:
                    cp.wait()

            def issue_gathers(sbuf, rows, sem):
                for j in range(P2R):
                    pltpu.async_copy(htab.at[sbuf.at[j]], rows.at[j], sem)

            def wait_gathers(sbuf, rows, sem):
                for j in range(P2R):
                    pltpu.make_async_copy(htab.at[sbuf.at[j]], rows.at[j],
                                          sem).wait()

            def scale_and_scatter(dbuf, abuf, rows):
                for j in range(P2R):
                    @plsc.parallel_loop(0, 128, unroll=8)
                    def _r(r):
                        sv = plsc.load_gather(abuf,
                                              [_splat16(j), _splat16(r)])
                        rows[j, r, pl.ds(0, 16)] = (
                            rows[j, r, pl.ds(0, 16)] * sv)
                        rows[j, r, pl.ds(16, 16)] = (
                            rows[j, r, pl.ds(16, 16)] * sv)
                for j in range(P2R):
                    pltpu.sync_copy(rows.at[j], oacc.at[dbuf.at[j]],
                                    add=True)

            base = sid * SL
            @pl.when(sid < NTI - 1)
            def _():
                pltpu.sync_copy(zeros_hbm.at[pl.ds(0, SL)],
                                oacc.at[pl.ds(base, SL)])
            @pl.when(sid == NTI - 1)
            def _():
                pltpu.sync_copy(zeros_hbm.at[pl.ds(0, SL_LAST)],
                                oacc.at[pl.ds(base, SL_LAST)])
            plsc.subcore_barrier()

            load_idx(0, src0, dst0, a0)
            issue_gathers(src0, rows0, sem0)

            @pl.loop(0, P2CHUNKS // 2)
            def _it(it):
                c0 = 2 * it
                load_idx(c0 + 1, src1, dst1, a1)
                issue_gathers(src1, rows1, sem1)
                wait_gathers(src0, rows0, sem0)
                scale_and_scatter(dst0, a0, rows0)
                @pl.when(c0 + 2 < P2CHUNKS)
                def _():
                    load_idx(c0 + 2, src0, dst0, a0)
                    issue_gathers(src0, rows0, sem0)
                wait_gathers(src1, rows1, sem1)
                scale_and_scatter(dst1, a1, rows1)

            plsc.subcore_barrier()
            opart_hbm = opart_list[h]
            for kk in range(24):
                pltpu.sync_copy(oacc.at[pl.ds(base + kk * 128, 128)],
                                rows0.at[0])
                pltpu.sync_copy(rows0.at[0],
                                opart_hbm.at[cid, pl.ds(base + kk * 128, 128)])
            @pl.when(sid < NTI - 1)
            def _():
                t = SL - 24 * 128
                pltpu.sync_copy(oacc.at[pl.ds(base + 24 * 128, t)],
                                rows0.at[0, pl.ds(0, t)])
                pltpu.sync_copy(rows0.at[0, pl.ds(0, t)],
                                opart_hbm.at[cid, pl.ds(base + 24 * 128, t)])
            @pl.when(sid == NTI - 1)
            def _():
                t = SL_LAST - 24 * 128
                pltpu.sync_copy(oacc.at[pl.ds(base + 24 * 128, t)],
                                rows0.at[0, pl.ds(0, t)])
                pltpu.sync_copy(rows0.at[0, pl.ds(0, t)],
                                opart_hbm.at[cid, pl.ds(base + 24 * 128, t)])
            plsc.subcore_barrier()

    k = pl.kernel(
        body,
        out_type=[jax.ShapeDtypeStruct((NSC, N, C), jnp.float32)
                  for _ in range(H)],
        mesh=_mesh,
        scratch_types=[
            pltpu.VMEM((P2R, 128), jnp.int32),
            pltpu.VMEM((P2R, 128), jnp.int32),
            pltpu.VMEM((P2R, 128), jnp.float32),
            pltpu.VMEM((P2R, 128, C), jnp.float32),
            pltpu.VMEM((P2R, 128), jnp.int32),
            pltpu.VMEM((P2R, 128), jnp.int32),
            pltpu.VMEM((P2R, 128), jnp.float32),
            pltpu.VMEM((P2R, 128, C), jnp.float32),
            pltpu.VMEM_SHARED((N, C), jnp.float32),
            pltpu.SemaphoreType.DMA,
            pltpu.SemaphoreType.DMA,
            pltpu.SemaphoreType.DMA,
        ],
        compiler_params=_cp,
    )
    return k(src2d, dst2d, alpha, *htabs, zeros_o)


def _embed_body(x_ref, w_ref, asf_ref, adf_ref,
                h0_ref, h1_ref, h2_ref, h3_ref, as_ref, ad_ref):
    hblk = jax.lax.dot_general(
        x_ref[...], w_ref[...], (((1,), (0,)), ((), ())),
        preferred_element_type=jnp.float32)
    for h, o in enumerate((h0_ref, h1_ref, h2_ref, h3_ref)):
        o[...] = hblk[:, h * C:(h + 1) * C]
    as_ref[...] = jax.lax.dot_general(
        asf_ref[...], hblk, (((1,), (1,)), ((), ())),
        preferred_element_type=jnp.float32)
    ad_ref[...] = jax.lax.dot_general(
        adf_ref[...], hblk, (((1,), (1,)), ((), ())),
        preferred_element_type=jnp.float32)


def _k_embed(hin, W, a_src, a_dst):
    din = hin.shape[1]
    eye4 = jnp.eye(H, dtype=jnp.float32)
    asf = (a_src[:, None, :] * eye4[:, :, None]).reshape(H, H * C)
    adf = (a_dst[:, None, :] * eye4[:, :, None]).reshape(H, H * C)
    outs = pl.pallas_call(
        _embed_body,
        grid=(NBLK,),
        in_specs=[
            pl.BlockSpec((BN, din), lambda i: (i, 0)),
            pl.BlockSpec((din, H * C), lambda i: (0, 0)),
            pl.BlockSpec((H, H * C), lambda i: (0, 0)),
            pl.BlockSpec((H, H * C), lambda i: (0, 0)),
        ],
        out_specs=[
            pl.BlockSpec((BN, C), lambda i: (i, 0)),
            pl.BlockSpec((BN, C), lambda i: (i, 0)),
            pl.BlockSpec((BN, C), lambda i: (i, 0)),
            pl.BlockSpec((BN, C), lambda i: (i, 0)),
            pl.BlockSpec((H, BN), lambda i: (0, i)),
            pl.BlockSpec((H, BN), lambda i: (0, i)),
        ],
        out_shape=[
            jax.ShapeDtypeStruct((N, C), jnp.float32),
            jax.ShapeDtypeStruct((N, C), jnp.float32),
            jax.ShapeDtypeStruct((N, C), jnp.float32),
            jax.ShapeDtypeStruct((N, C), jnp.float32),
            jax.ShapeDtypeStruct((H, N), jnp.float32),
            jax.ShapeDtypeStruct((H, N), jnp.float32),
        ],
    )(hin, W, asf, adf)
    return outs[:4], outs[4], outs[5]


def _dsum_body(d_ref, o_ref):
    o_ref[...] = d_ref[0] + d_ref[1]


def _k_dsum(dpart):
    return pl.pallas_call(
        _dsum_body,
        grid=(NBLK,),
        in_specs=[pl.BlockSpec((NSC, H, BN), lambda i: (0, 0, i))],
        out_specs=pl.BlockSpec((H, BN), lambda i: (0, i)),
        out_shape=jax.ShapeDtypeStruct((H, N), jnp.float32),
    )(dpart)


def _comb_body(p0_ref, p1_ref, p2_ref, p3_ref, d_ref, s_ref, b_ref,
               h_ref, st_ref, ssum, ssq):
    i = pl.program_id(0)
    dblk = d_ref[0] + d_ref[1]
    den_exp = jax.lax.dot_general(
        dblk, s_ref[...], (((0,), (0,)), ((), ())),
        preferred_element_type=jnp.float32) + 1e-16
    hpre = jnp.concatenate(
        [p[0] + p[1] for p in (p0_ref, p1_ref, p2_ref, p3_ref)],
        axis=1) / den_exp + b_ref[...]
    h_ref[...] = hpre
    valid = (i * BN + lax.broadcasted_iota(jnp.int32, (BN, 1), 0)) < N
    hm = jnp.where(valid, hpre, 0.0)

    @pl.when(i == 0)
    def _():
        ssum[...] = jnp.zeros_like(ssum)
        ssq[...] = jnp.zeros_like(ssq)

    ssum[0:1, :] += jnp.sum(hm, axis=0, keepdims=True)
    ssq[0:1, :] += jnp.sum(hm * hm, axis=0, keepdims=True)

    @pl.when(i == NBLK - 1)
    def _():
        mu = ssum[0:1, :] / N
        var = ssq[0:1, :] / N - mu * mu
        st_ref[...] = jnp.concatenate(
            [mu, var, jnp.zeros((6, HID), jnp.float32)], axis=0)


def _k_comb(oparts, dpart, b):
    sel = (jnp.eye(H, dtype=jnp.float32)[:, :, None]
           * jnp.ones((1, 1, C), jnp.float32)).reshape(H, HID)
    return pl.pallas_call(
        _comb_body,
        grid=(NBLK,),
        in_specs=[
            pl.BlockSpec((NSC, BN, C), lambda i: (0, i, 0)),
            pl.BlockSpec((NSC, BN, C), lambda i: (0, i, 0)),
            pl.BlockSpec((NSC, BN, C), lambda i: (0, i, 0)),
            pl.BlockSpec((NSC, BN, C), lambda i: (0, i, 0)),
            pl.BlockSpec((NSC, H, BN), lambda i: (0, 0, i)),
            pl.BlockSpec((H, HID), lambda i: (0, 0)),
            pl.BlockSpec((1, HID), lambda i: (0, 0)),
        ],
        out_specs=[
            pl.BlockSpec((BN, HID), lambda i: (i, 0)),
            pl.BlockSpec((8, HID), lambda i: (0, 0)),
        ],
        out_shape=[
            jax.ShapeDtypeStruct((N, HID), jnp.float32),
            jax.ShapeDtypeStruct((8, HID), jnp.float32),
        ],
        scratch_shapes=[
            pltpu.VMEM((8, HID), jnp.float32),
            pltpu.VMEM((8, HID), jnp.float32),
        ],
    )(*oparts, dpart, sel, b.reshape(1, HID))


def _norm_body(h_ref, st_ref, g_ref, be_ref, o_ref):
    mu = st_ref[0:1, :]
    var = st_ref[1:2, :]
    xn = g_ref[...] * (h_ref[...] - mu) * lax.rsqrt(var + EPS_BN) + be_ref[...]
    o_ref[...] = jnp.maximum(xn, 0.0)


def _k_norm(hpre, stats, g, be):
    return pl.pallas_call(
        _norm_body,
        grid=(NBLK,),
        in_specs=[
            pl.BlockSpec((BN, HID), lambda i: (i, 0)),
            pl.BlockSpec((8, HID), lambda i: (0, 0)),
            pl.BlockSpec((1, HID), lambda i: (0, 0)),
            pl.BlockSpec((1, HID), lambda i: (0, 0)),
        ],
        out_specs=pl.BlockSpec((BN, HID), lambda i: (i, 0)),
        out_shape=jax.ShapeDtypeStruct((N, HID), jnp.float32),
    )(hpre, stats, g.reshape(1, HID), be.reshape(1, HID))


def _pool_body(h_ref, b_ref, bs_ref, wm_ref, wx_ref, bo_ref, o_ref,
               macc, cacc, xacc):
    i = pl.program_id(0)

    @pl.when(i == 0)
    def _():
        macc[...] = jnp.zeros_like(macc)
        cacc[...] = jnp.zeros_like(cacc)
        xacc[...] = jnp.full_like(xacc, -jnp.inf)

    hblk = h_ref[...]
    bl = b_ref[0]
    bs = bs_ref[0]
    valid_l = (i * BN + lax.broadcasted_iota(jnp.int32, (1, BN), 1)) < N
    valid_s = (i * BN + lax.broadcasted_iota(jnp.int32, (BN, 1), 0)) < N
    gid = lax.broadcasted_iota(jnp.int32, (G, BN), 0)
    onehot = jnp.where((bl == gid) & valid_l, 1.0, 0.0)
    macc[...] += jax.lax.dot_general(
        onehot, hblk, (((1,), (0,)), ((), ())),
        preferred_element_type=jnp.float32)
    cacc[...] += jax.lax.dot_general(
        onehot, jnp.ones((BN, HID), jnp.float32), (((1,), (0,)), ((), ())),
        preferred_element_type=jnp.float32)
    bmin = jnp.min(jnp.where(valid_l, bl, G))
    bmax = jnp.max(jnp.where(valid_l, bl, -1))
    for g in range(G):
        @pl.when((g >= bmin) & (g <= bmax))
        def _():
            sel = (bs == g) & valid_s
            mg = jnp.max(jnp.where(sel, hblk, -jnp.inf),
                         axis=0, keepdims=True)
            xacc[g:g + 1, :] = jnp.maximum(xacc[g:g + 1, :], mg)

    @pl.when(i == NBLK - 1)
    def _():
        gmean = macc[...] / jnp.maximum(cacc[...], 1.0)
        gmax = xacc[...]
        gmax = jnp.where(gmax == -jnp.inf, 0.0, gmax)
        o_ref[...] = (
            jax.lax.dot_general(gmean, wm_ref[...], (((1,), (0,)), ((), ())),
                                preferred_element_type=jnp.float32)
            + jax.lax.dot_general(gmax, wx_ref[...], (((1,), (0,)), ((), ())),
                                  preferred_element_type=jnp.float32)
            + bo_ref[...])


def _k_pool(hn, batch3d, batch_sub, Wout, bout):
    return pl.pallas_call(
        _pool_body,
        grid=(NBLK,),
        in_specs=[
            pl.BlockSpec((BN, HID), lambda i: (i, 0)),
            pl.BlockSpec((1, 1, BN), lambda i: (i, 0, 0)),
            pl.BlockSpec((1, BN, 1), lambda i: (i, 0, 0)),
            pl.BlockSpec((HID, HID), lambda i: (0, 0)),
            pl.BlockSpec((HID, HID), lambda i: (0, 0)),
            pl.BlockSpec((1, HID), lambda i: (0, 0)),
        ],
        out_specs=pl.BlockSpec((G, HID), lambda i: (0, 0)),
        out_shape=jax.ShapeDtypeStruct((G, HID), jnp.float32),
        scratch_shapes=[
            pltpu.VMEM((G, HID), jnp.float32),
            pltpu.VMEM((G, HID), jnp.float32),
            pltpu.VMEM((G, HID), jnp.float32),
        ],
    )(hn, batch3d, batch_sub, Wout[:HID], Wout[HID:], bout.reshape(1, HID))


def kernel(x, edge_index, batch, W1, a_src1, a_dst1, b1, g1, be1,
           W2, a_src2, a_dst2, b2, g2, be2, W3, a_src3, a_dst3, b3, g3, be3,
           Wout, bout):
    src = jnp.pad(edge_index[0], (0, EPAD - E)).reshape(ROWS, 128)
    dst = jnp.concatenate(
        [edge_index[1], jnp.arange(EPAD - E, dtype=jnp.int32) % N]
    ).reshape(ROWS, 128)
    zeros1 = jnp.zeros((3200,), jnp.float32)
    zeros_o = jnp.zeros((SL, C), jnp.float32)
    bpad = jnp.pad(batch, (0, NPAD - N), constant_values=G)
    batch3d = bpad.reshape(NBLK, 1, BN)
    batch_sub = bpad.reshape(NBLK, BN, 1)

    h = x
    for (W, a_s, a_d, b, g, be) in (
            (W1, a_src1, a_dst1, b1, g1, be1),
            (W2, a_src2, a_dst2, b2, g2, be2),
            (W3, a_src3, a_dst3, b3, g3, be3)):
        htabs, asv, adv = _k_embed(h, W, a_s, a_d)
        w, dpart = _sc_pass1(src, dst, asv.reshape(-1), adv.reshape(-1),
                             zeros1)
        oparts = _sc_pass2(src, dst, w, htabs, zeros_o)
        hpre, stats = _k_comb(oparts, dpart.reshape(NSC, H, N), b)
        h = _k_norm(hpre, stats, g, be)

    return _k_pool(h, batch3d, batch_sub, Wout, bout)


# speedup vs baseline: 52.0795x; 1.0352x over previous
import dataclasses
import functools
import jax
import jax.numpy as jnp
from jax import lax
from jax.experimental import pallas as pl
from jax.experimental.pallas import tpu as pltpu
from jax.experimental.pallas import tpu_sc as plsc

N = 50000
E = 800000
H = 4
C = 32
HID = 128
G = 64
NEG_SLOPE = 0.2
EPS_BN = 1e-5

NSC = 2
NTI = 16
NW = NSC * NTI
EPT = 25600
EPAD = NW * EPT
ROWS = EPAD // 128
RPT = EPT // 128
CHUNK_ROWS = 8
NCHUNK = RPT // CHUNK_ROWS
SL = 3128
SL_LAST = N - 15 * SL

BN = 512
NBLK = (N + BN - 1) // BN
NPAD = NBLK * BN

_mesh = plsc.VectorSubcoreMesh(core_axis_name="c", subcore_axis_name="s")

_cp = pltpu.CompilerParams()
if "needs_layout_passes" in pltpu.CompilerParams.__dataclass_fields__:
    _cp = dataclasses.replace(_cp, needs_layout_passes=False)
if "use_tc_tiling_on_sc" in pltpu.CompilerParams.__dataclass_fields__:
    _cp = dataclasses.replace(_cp, use_tc_tiling_on_sc=False)


def _iota16():
    return lax.iota(jnp.int32, 16)


def _splat16(v):
    return lax.broadcast(v, (16,))


def _sc_pass1(src2d, dst2d, asv, adv, zeros1):
    def body(src_hbm, dst_hbm, as_hbm, ad_hbm, zeros_hbm, w_hbm, dpart_hbm,
             as_tab, ad_tab, src_buf, dst_buf, w_buf, dzero, dstage, dacc,
             sem):
        cid = lax.axis_index("c")
        sid = lax.axis_index("s")
        wid = cid * NTI + sid
        tr0 = wid * RPT

        pltpu.sync_copy(zeros_hbm.at[pl.ds(0, 3200)], dzero)

        for h in range(H):
            pltpu.sync_copy(as_hbm.at[pl.ds(h * N, N)], as_tab)
            pltpu.sync_copy(ad_hbm.at[pl.ds(h * N, N)], ad_tab)
            @pl.when(sid < NTI - 1)
            def _():
                pltpu.sync_copy(dzero.at[pl.ds(0, SL)],
                                dacc.at[pl.ds(sid * SL, SL)])
            @pl.when(sid == NTI - 1)
            def _():
                pltpu.sync_copy(dzero.at[pl.ds(0, SL_LAST)],
                                dacc.at[pl.ds(sid * SL, SL_LAST)])
            plsc.subcore_barrier()

            @pl.loop(0, NCHUNK)
            def _chunk(ch):
                r0 = tr0 + ch * CHUNK_ROWS
                cps = [
                    pltpu.async_copy(src_hbm.at[pl.ds(r0, CHUNK_ROWS)],
                                     src_buf, sem),
                    pltpu.async_copy(dst_hbm.at[pl.ds(r0, CHUNK_ROWS)],
                                     dst_buf, sem),
                ]
                for cp in cps:
                    cp.wait()

                @plsc.parallel_loop(0, CHUNK_ROWS * 8, unroll=4)
                def _grp(g):
                    row = g >> 3
                    off = (g & 7) * 16
                    sv = src_buf[row, pl.ds(off, 16)]
                    dv = dst_buf[row, pl.ds(off, 16)]
                    es = plsc.load_gather(as_tab, [sv])
                    ed = plsc.load_gather(ad_tab, [dv])
                    e = es + ed
                    e = jnp.where(e > 0, e, NEG_SLOPE * e)
                    wv = jnp.exp(e)
                    eid = _splat16(r0 * 128 + g * 16) + _iota16()
                    wv = jnp.where(eid < E, wv, 0.0)
                    w_buf[row, pl.ds(off, 16)] = wv

                pltpu.sync_copy(w_buf, w_hbm.at[h, pl.ds(r0, CHUNK_ROWS)])
                for j in range(CHUNK_ROWS):
                    pltpu.sync_copy(w_buf.at[j], dacc.at[dst_buf.at[j]],
                                    add=True)

            plsc.subcore_barrier()
            dbase = cid * (H * N) + h * N + sid * SL
            @pl.when(sid < NTI - 1)
            def _():
                pltpu.sync_copy(dacc.at[pl.ds(sid * SL, SL)],
                                dstage.at[pl.ds(0, SL)])
                pltpu.sync_copy(dstage.at[pl.ds(0, SL)],
                                dpart_hbm.at[pl.ds(dbase, SL)])
            @pl.when(sid == NTI - 1)
            def _():
                pltpu.sync_copy(dacc.at[pl.ds(sid * SL, SL_LAST)],
                                dstage.at[pl.ds(0, SL_LAST)])
                pltpu.sync_copy(dstage.at[pl.ds(0, SL_LAST)],
                                dpart_hbm.at[pl.ds(dbase, SL_LAST)])
            plsc.subcore_barrier()

    k = pl.kernel(
        body,
        out_type=[
            jax.ShapeDtypeStruct((H, ROWS, 128), jnp.float32),
            jax.ShapeDtypeStruct((NSC * H * N,), jnp.float32),
        ],
        mesh=_mesh,
        scratch_types=[
            pltpu.VMEM((N,), jnp.float32),
            pltpu.VMEM((N,), jnp.float32),
            pltpu.VMEM((CHUNK_ROWS, 128), jnp.int32),
            pltpu.VMEM((CHUNK_ROWS, 128), jnp.int32),
            pltpu.VMEM((CHUNK_ROWS, 128), jnp.float32),
            pltpu.VMEM((3200,), jnp.float32),
            pltpu.VMEM((SL,), jnp.float32),
            pltpu.VMEM_SHARED((N,), jnp.float32),
            pltpu.SemaphoreType.DMA,
        ],
        compiler_params=_cp,
    )
    return k(src2d, dst2d, asv, adv, zeros1)


def _sc_alpha(dst2d, w, denom):
    def body(dst_hbm, w_hbm, den_hbm, a_hbm,
             den_tab, dst_buf, w_buf, a_buf, sem):
        cid = lax.axis_index("c")
        sid = lax.axis_index("s")
        wid = cid * NTI + sid
        tr0 = wid * RPT

        for h in range(H):
            pltpu.sync_copy(den_hbm.at[pl.ds(h * N, N)], den_tab)

            @pl.loop(0, RPT // 8)
            def _chunk(ch):
                r0 = tr0 + ch * 8
                pltpu.sync_copy(dst_hbm.at[pl.ds(r0, 8)], dst_buf)
                pltpu.sync_copy(w_hbm.at[h, pl.ds(r0, 8)], w_buf)

                @plsc.parallel_loop(0, 64, unroll=4)
                def _grp(g):
                    row = g >> 3
                    off = (g & 7) * 16
                    dv_idx = dst_buf[row, pl.ds(off, 16)]
                    dvals = plsc.load_gather(den_tab, [dv_idx])
                    wv = w_buf[row, pl.ds(off, 16)]
                    a_buf[row, pl.ds(off, 16)] = wv / (dvals + 1e-16)

                pltpu.sync_copy(a_buf, a_hbm.at[h, pl.ds(r0, 8)])

    k = pl.kernel(
        body,
        out_type=jax.ShapeDtypeStruct((H, ROWS, 128), jnp.float32),
        mesh=_mesh,
        scratch_types=[
            pltpu.VMEM((N,), jnp.float32),
            pltpu.VMEM((8, 128), jnp.int32),
            pltpu.VMEM((8, 128), jnp.float32),
            pltpu.VMEM((8, 128), jnp.float32),
            pltpu.SemaphoreType.DMA,
        ],
        compiler_params=_cp,
    )
    return k(dst2d, w, denom)


P2R = 2
P2CHUNKS = RPT // P2R


def _sc_pass2(src2d, dst2d, alpha, htabs, zeros_o):
    def body(src_hbm, dst_hbm, a_hbm, h0, h1, h2, h3, zeros_hbm,
             op0, op1, op2, op3,
             src0, dst0, a0, rows0, src1, dst1, a1, rows1, oacc,
             sem0, sem1, semi):
        opart_list = [op0, op1, op2, op3]
        cid = lax.axis_index("c")
        sid = lax.axis_index("s")
        wid = cid * NTI + sid
        tr0 = wid * RPT
        htab_list = [h0, h1, h2, h3]
        sets = ((src0, dst0, a0, rows0, sem0), (src1, dst1, a1, rows1, sem1))

        for h in range(H):
            htab = htab_list[h]

            def load_idx(c, sbuf, dbuf, abuf):
                r0 = tr0 + c * P2R
                cps = [
                    pltpu.async_copy(src_hbm.at[pl.ds(r0, P2R)], sbuf, semi),
                    pltpu.async_copy(dst_hbm.at[pl.ds(r0, P2R)], dbuf, semi),
                    pltpu.async_copy(a_hbm.at[h, pl.ds(r0, P2R)], abuf, semi),
                ]
                for cp in cps:
                    cp.wait()

            def issue_gathers(sbuf, rows, sem):
                for j in range(P2R):
                    pltpu.async_copy(htab.at[sbuf.at[j]], rows.at[j], sem)

            def wait_gathers(sbuf, rows, sem):
                for j in range(P2R):
                    pltpu.make_async_copy(htab.at[sbuf.at[j]], rows.at[j],
                                          sem).wait()

            def scale_and_scatter(dbuf, abuf, rows):
                for j in range(P2R):
                    @plsc.parallel_loop(0, 128, unroll=8)
                    def _r(r):
                        sv = plsc.load_gather(abuf,
                                              [_splat16(j), _splat16(r)])
                        rows[j, r, pl.ds(0, 16)] = (
                            rows[j, r, pl.ds(0, 16)] * sv)
                        rows[j, r, pl.ds(16, 16)] = (
                            rows[j, r, pl.ds(16, 16)] * sv)
                for j in range(P2R):
                    pltpu.sync_copy(rows.at[j], oacc.at[dbuf.at[j]],
                                    add=True)

            base = sid * SL
            @pl.when(sid < NTI - 1)
            def _():
                pltpu.sync_copy(zeros_hbm.at[pl.ds(0, SL)],
                                oacc.at[pl.ds(base, SL)])
            @pl.when(sid == NTI - 1)
            def _():
                pltpu.sync_copy(zeros_hbm.at[pl.ds(0, SL_LAST)],
                                oacc.at[pl.ds(base, SL_LAST)])
            plsc.subcore_barrier()

            load_idx(0, src0, dst0, a0)
            issue_gathers(src0, rows0, sem0)

            @pl.loop(0, P2CHUNKS // 2)
            def _it(it):
                c0 = 2 * it
                load_idx(c0 + 1, src1, dst1, a1)
                issue_gathers(src1, rows1, sem1)
                wait_gathers(src0, rows0, sem0)
                scale_and_scatter(dst0, a0, rows0)
                @pl.when(c0 + 2 < P2CHUNKS)
                def _():
                    load_idx(c0 + 2, src0, dst0, a0)
                    issue_gathers(src0, rows0, sem0)
                wait_gathers(src1, rows1, sem1)
                scale_and_scatter(dst1, a1, rows1)

            plsc.subcore_barrier()
            opart_hbm = opart_list[h]
            for kk in range(24):
                pltpu.sync_copy(oacc.at[pl.ds(base + kk * 128, 128)],
                                rows0.at[0])
                pltpu.sync_copy(rows0.at[0],
                                opart_hbm.at[cid, pl.ds(base + kk * 128, 128)])
            @pl.when(sid < NTI - 1)
            def _():
                t = SL - 24 * 128
                pltpu.sync_copy(oacc.at[pl.ds(base + 24 * 128, t)],
                                rows0.at[0, pl.ds(0, t)])
                pltpu.sync_copy(rows0.at[0, pl.ds(0, t)],
                                opart_hbm.at[cid, pl.ds(base + 24 * 128, t)])
            @pl.when(sid == NTI - 1)
            def _():
                t = SL_LAST - 24 * 128
                pltpu.sync_copy(oacc.at[pl.ds(base + 24 * 128, t)],
                                rows0.at[0, pl.ds(0, t)])
                pltpu.sync_copy(rows0.at[0, pl.ds(0, t)],
                                opart_hbm.at[cid, pl.ds(base + 24 * 128, t)])
            plsc.subcore_barrier()

    k = pl.kernel(
        body,
        out_type=[jax.ShapeDtypeStruct((NSC, N, C), jnp.float32)
                  for _ in range(H)],
        mesh=_mesh,
        scratch_types=[
            pltpu.VMEM((P2R, 128), jnp.int32),
            pltpu.VMEM((P2R, 128), jnp.int32),
            pltpu.VMEM((P2R, 128), jnp.float32),
            pltpu.VMEM((P2R, 128, C), jnp.float32),
            pltpu.VMEM((P2R, 128), jnp.int32),
            pltpu.VMEM((P2R, 128), jnp.int32),
            pltpu.VMEM((P2R, 128), jnp.float32),
            pltpu.VMEM((P2R, 128, C), jnp.float32),
            pltpu.VMEM_SHARED((N, C), jnp.float32),
            pltpu.SemaphoreType.DMA,
            pltpu.SemaphoreType.DMA,
            pltpu.SemaphoreType.DMA,
        ],
        compiler_params=_cp,
    )
    return k(src2d, dst2d, alpha, *htabs, zeros_o)


def _embed_body(x_ref, w_ref, asf_ref, adf_ref,
                h0_ref, h1_ref, h2_ref, h3_ref, as_ref, ad_ref):
    hblk = jax.lax.dot_general(
        x_ref[...], w_ref[...], (((1,), (0,)), ((), ())),
        preferred_element_type=jnp.float32)
    _embed_tail(hblk, h0_ref, h1_ref, h2_ref, h3_ref, as_ref, ad_ref,
                asf_ref, adf_ref)


def _embed_bn_body(x_ref, st_ref, g_ref, be_ref, w_ref, asf_ref, adf_ref,
                   h0_ref, h1_ref, h2_ref, h3_ref, as_ref, ad_ref):
    mu = st_ref[0:1, :]
    var = st_ref[1:2, :]
    xn = (g_ref[...] * (x_ref[...] - mu) * lax.rsqrt(var + EPS_BN)
          + be_ref[...])
    xn = jnp.maximum(xn, 0.0)
    hblk = jax.lax.dot_general(
        xn, w_ref[...], (((1,), (0,)), ((), ())),
        preferred_element_type=jnp.float32)
    _embed_tail(hblk, h0_ref, h1_ref, h2_ref, h3_ref, as_ref, ad_ref,
                asf_ref, adf_ref)


def _embed_tail(hblk, h0_ref, h1_ref, h2_ref, h3_ref, as_ref, ad_ref,
                asf_ref, adf_ref):
    for h, o in enumerate((h0_ref, h1_ref, h2_ref, h3_ref)):
        o[...] = hblk[:, h * C:(h + 1) * C]
    as_ref[...] = jax.lax.dot_general(
        asf_ref[...], hblk, (((1,), (1,)), ((), ())),
        preferred_element_type=jnp.float32)
    ad_ref[...] = jax.lax.dot_general(
        adf_ref[...], hblk, (((1,), (1,)), ((), ())),
        preferred_element_type=jnp.float32)


def _k_embed(hin, W, a_src, a_dst, stats=None, g=None, be=None):
    din = hin.shape[1]
    eye4 = jnp.eye(H, dtype=jnp.float32)
    asf = (a_src[:, None, :] * eye4[:, :, None]).reshape(H, H * C)
    adf = (a_dst[:, None, :] * eye4[:, :, None]).reshape(H, H * C)
    if stats is None:
        body = _embed_body
        extra_in = []
        extra_specs = []
    else:
        body = _embed_bn_body
        extra_in = [stats, g.reshape(1, HID), be.reshape(1, HID)]
        extra_specs = [
            pl.BlockSpec((8, HID), lambda i: (0, 0)),
            pl.BlockSpec((1, HID), lambda i: (0, 0)),
            pl.BlockSpec((1, HID), lambda i: (0, 0)),
        ]
    outs = pl.pallas_call(
        body,
        grid=(NBLK,),
        in_specs=[
            pl.BlockSpec((BN, din), lambda i: (i, 0)),
        ] + extra_specs + [
            pl.BlockSpec((din, H * C), lambda i: (0, 0)),
            pl.BlockSpec((H, H * C), lambda i: (0, 0)),
            pl.BlockSpec((H, H * C), lambda i: (0, 0)),
        ],
        out_specs=[
            pl.BlockSpec((BN, C), lambda i: (i, 0)),
            pl.BlockSpec((BN, C), lambda i: (i, 0)),
            pl.BlockSpec((BN, C), lambda i: (i, 0)),
            pl.BlockSpec((BN, C), lambda i: (i, 0)),
            pl.BlockSpec((H, BN), lambda i: (0, i)),
            pl.BlockSpec((H, BN), lambda i: (0, i)),
        ],
        out_shape=[
            jax.ShapeDtypeStruct((N, C), jnp.float32),
            jax.ShapeDtypeStruct((N, C), jnp.float32),
            jax.ShapeDtypeStruct((N, C), jnp.float32),
            jax.ShapeDtypeStruct((N, C), jnp.float32),
            jax.ShapeDtypeStruct((H, N), jnp.float32),
            jax.ShapeDtypeStruct((H, N), jnp.float32),
        ],
    )(hin, *extra_in, W, asf, adf)
    return outs[:4], outs[4], outs[5]


def _dsum_body(d_ref, o_ref):
    o_ref[...] = d_ref[0] + d_ref[1]


def _k_dsum(dpart):
    return pl.pallas_call(
        _dsum_body,
        grid=(NBLK,),
        in_specs=[pl.BlockSpec((NSC, H, BN), lambda i: (0, 0, i))],
        out_specs=pl.BlockSpec((H, BN), lambda i: (0, i)),
        out_shape=jax.ShapeDtypeStruct((H, N), jnp.float32),
    )(dpart)


def _comb_body(p0_ref, p1_ref, p2_ref, p3_ref, d_ref, s_ref, b_ref,
               h_ref, st_ref, ssum, ssq):
    i = pl.program_id(0)
    dblk = d_ref[0] + d_ref[1]
    den_exp = jax.lax.dot_general(
        dblk, s_ref[...], (((0,), (0,)), ((), ())),
        preferred_element_type=jnp.float32) + 1e-16
    hpre = jnp.concatenate(
        [p[0] + p[1] for p in (p0_ref, p1_ref, p2_ref, p3_ref)],
        axis=1) / den_exp + b_ref[...]
    h_ref[...] = hpre
    valid = (i * BN + lax.broadcasted_iota(jnp.int32, (BN, 1), 0)) < N
    hm = jnp.where(valid, hpre, 0.0)

    @pl.when(i == 0)
    def _():
        ssum[...] = jnp.zeros_like(ssum)
        ssq[...] = jnp.zeros_like(ssq)

    ssum[0:1, :] += jnp.sum(hm, axis=0, keepdims=True)
    ssq[0:1, :] += jnp.sum(hm * hm, axis=0, keepdims=True)

    @pl.when(i == NBLK - 1)
    def _():
        mu = ssum[0:1, :] / N
        var = ssq[0:1, :] / N - mu * mu
        st_ref[...] = jnp.concatenate(
            [mu, var, jnp.zeros((6, HID), jnp.float32)], axis=0)


def _k_comb(oparts, dpart, b):
    sel = (jnp.eye(H, dtype=jnp.float32)[:, :, None]
           * jnp.ones((1, 1, C), jnp.float32)).reshape(H, HID)
    return pl.pallas_call(
        _comb_body,
        grid=(NBLK,),
        in_specs=[
            pl.BlockSpec((NSC, BN, C), lambda i: (0, i, 0)),
            pl.BlockSpec((NSC, BN, C), lambda i: (0, i, 0)),
            pl.BlockSpec((NSC, BN, C), lambda i: (0, i, 0)),
            pl.BlockSpec((NSC, BN, C), lambda i: (0, i, 0)),
            pl.BlockSpec((NSC, H, BN), lambda i: (0, 0, i)),
            pl.BlockSpec((H, HID), lambda i: (0, 0)),
            pl.BlockSpec((1, HID), lambda i: (0, 0)),
        ],
        out_specs=[
            pl.BlockSpec((BN, HID), lambda i: (i, 0)),
            pl.BlockSpec((8, HID), lambda i: (0, 0)),
        ],
        out_shape=[
            jax.ShapeDtypeStruct((N, HID), jnp.float32),
            jax.ShapeDtypeStruct((8, HID), jnp.float32),
        ],
        scratch_shapes=[
            pltpu.VMEM((8, HID), jnp.float32),
            pltpu.VMEM((8, HID), jnp.float32),
        ],
    )(*oparts, dpart, sel, b.reshape(1, HID))


def _norm_body(h_ref, st_ref, g_ref, be_ref, o_ref):
    mu = st_ref[0:1, :]
    var = st_ref[1:2, :]
    xn = g_ref[...] * (h_ref[...] - mu) * lax.rsqrt(var + EPS_BN) + be_ref[...]
    o_ref[...] = jnp.maximum(xn, 0.0)


def _k_norm(hpre, stats, g, be):
    return pl.pallas_call(
        _norm_body,
        grid=(NBLK,),
        in_specs=[
            pl.BlockSpec((BN, HID), lambda i: (i, 0)),
            pl.BlockSpec((8, HID), lambda i: (0, 0)),
            pl.BlockSpec((1, HID), lambda i: (0, 0)),
            pl.BlockSpec((1, HID), lambda i: (0, 0)),
        ],
        out_specs=pl.BlockSpec((BN, HID), lambda i: (i, 0)),
        out_shape=jax.ShapeDtypeStruct((N, HID), jnp.float32),
    )(hpre, stats, g.reshape(1, HID), be.reshape(1, HID))


def _pool_body(h_ref, st_ref, g_ref, be_ref, b_ref, bs_ref, wm_ref,
               wx_ref, bo_ref, o_ref, macc, cacc, xacc):
    i = pl.program_id(0)

    @pl.when(i == 0)
    def _():
        macc[...] = jnp.zeros_like(macc)
        cacc[...] = jnp.zeros_like(cacc)
        xacc[...] = jnp.full_like(xacc, -jnp.inf)

    mu = st_ref[0:1, :]
    var = st_ref[1:2, :]
    hblk = (g_ref[...] * (h_ref[...] - mu) * lax.rsqrt(var + EPS_BN)
            + be_ref[...])
    hblk = jnp.maximum(hblk, 0.0)
    bl = b_ref[0]
    bs = bs_ref[0]
    valid_l = (i * BN + lax.broadcasted_iota(jnp.int32, (1, BN), 1)) < N
    valid_s = (i * BN + lax.broadcasted_iota(jnp.int32, (BN, 1), 0)) < N
    gid = lax.broadcasted_iota(jnp.int32, (G, BN), 0)
    onehot = jnp.where((bl == gid) & valid_l, 1.0, 0.0)
    macc[...] += jax.lax.dot_general(
        onehot, hblk, (((1,), (0,)), ((), ())),
        preferred_element_type=jnp.float32)
    cacc[...] += jax.lax.dot_general(
        onehot, jnp.ones((BN, HID), jnp.float32), (((1,), (0,)), ((), ())),
        preferred_element_type=jnp.float32)
    bmin = jnp.min(jnp.where(valid_l, bl, G))
    bmax = jnp.max(jnp.where(valid_l, bl, -1))
    for g in range(G):
        @pl.when((g >= bmin) & (g <= bmax))
        def _():
            sel = (bs == g) & valid_s
            mg = jnp.max(jnp.where(sel, hblk, -jnp.inf),
                         axis=0, keepdims=True)
            xacc[g:g + 1, :] = jnp.maximum(xacc[g:g + 1, :], mg)

    @pl.when(i == NBLK - 1)
    def _():
        gmean = macc[...] / jnp.maximum(cacc[...], 1.0)
        gmax = xacc[...]
        gmax = jnp.where(gmax == -jnp.inf, 0.0, gmax)
        o_ref[...] = (
            jax.lax.dot_general(gmean, wm_ref[...], (((1,), (0,)), ((), ())),
                                preferred_element_type=jnp.float32)
            + jax.lax.dot_general(gmax, wx_ref[...], (((1,), (0,)), ((), ())),
                                  preferred_element_type=jnp.float32)
            + bo_ref[...])


def _k_pool(hn, stats, g, be, batch3d, batch_sub, Wout, bout):
    return pl.pallas_call(
        _pool_body,
        grid=(NBLK,),
        in_specs=[
            pl.BlockSpec((BN, HID), lambda i: (i, 0)),
            pl.BlockSpec((8, HID), lambda i: (0, 0)),
            pl.BlockSpec((1, HID), lambda i: (0, 0)),
            pl.BlockSpec((1, HID), lambda i: (0, 0)),
            pl.BlockSpec((1, 1, BN), lambda i: (i, 0, 0)),
            pl.BlockSpec((1, BN, 1), lambda i: (i, 0, 0)),
            pl.BlockSpec((HID, HID), lambda i: (0, 0)),
            pl.BlockSpec((HID, HID), lambda i: (0, 0)),
            pl.BlockSpec((1, HID), lambda i: (0, 0)),
        ],
        out_specs=pl.BlockSpec((G, HID), lambda i: (0, 0)),
        out_shape=jax.ShapeDtypeStruct((G, HID), jnp.float32),
        scratch_shapes=[
            pltpu.VMEM((G, HID), jnp.float32),
            pltpu.VMEM((G, HID), jnp.float32),
            pltpu.VMEM((G, HID), jnp.float32),
        ],
    )(hn, stats, g.reshape(1, HID), be.reshape(1, HID), batch3d,
      batch_sub, Wout[:HID], Wout[HID:], bout.reshape(1, HID))


def kernel(x, edge_index, batch, W1, a_src1, a_dst1, b1, g1, be1,
           W2, a_src2, a_dst2, b2, g2, be2, W3, a_src3, a_dst3, b3, g3, be3,
           Wout, bout):
    src = jnp.pad(edge_index[0], (0, EPAD - E)).reshape(ROWS, 128)
    dst = jnp.concatenate(
        [edge_index[1], jnp.arange(EPAD - E, dtype=jnp.int32) % N]
    ).reshape(ROWS, 128)
    zeros1 = jnp.zeros((3200,), jnp.float32)
    zeros_o = jnp.zeros((SL, C), jnp.float32)
    bpad = jnp.pad(batch, (0, NPAD - N), constant_values=G)
    batch3d = bpad.reshape(NBLK, 1, BN)
    batch_sub = bpad.reshape(NBLK, BN, 1)

    h = x
    prev = None
    for (W, a_s, a_d, b, g, be) in (
            (W1, a_src1, a_dst1, b1, g1, be1),
            (W2, a_src2, a_dst2, b2, g2, be2),
            (W3, a_src3, a_dst3, b3, g3, be3)):
        if prev is None:
            htabs, asv, adv = _k_embed(h, W, a_s, a_d)
        else:
            htabs, asv, adv = _k_embed(h, W, a_s, a_d, *prev)
        w, dpart = _sc_pass1(src, dst, asv.reshape(-1), adv.reshape(-1),
                             zeros1)
        oparts = _sc_pass2(src, dst, w, htabs, zeros_o)
        h, stats = _k_comb(oparts, dpart.reshape(NSC, H, N), b)
        prev = (stats, g, be)

    return _k_pool(h, *prev, batch3d, batch_sub, Wout, bout)
